# Initial kernel scaffold; baseline (speedup 1.0000x reference)
import jax
import jax.numpy as jnp
from jax.experimental import pallas as pl


def kernel(x, edge_index, batch, w1_on, b1_on, w1_off, b1_off, w2_on, b2_on, w2_off, b2_off, w3_on, b3_on, w3_off, b3_off, wl1, bl1, wl2, bl2):
    raise NotImplementedError("write your pallas kernel here")


# speedup vs baseline: 1.8124x; 1.8124x over previous
import functools
import jax
import jax.numpy as jnp
from jax.experimental import pallas as pl
from jax.experimental.pallas import tpu as pltpu

N = 10000
H = 128
NPAD = 10240


def _node_mm_kernel(x_ref, won_ref, bon_ref, woff_ref, boff_ref, u_ref, v_ref):
    x = x_ref[...]
    u_ref[...] = x @ won_ref[...] + bon_ref[...]
    v_ref[...] = x @ woff_ref[...] + boff_ref[...]


def _dual_matmul(x, w_on, b_on, w_off, b_off, tile=512):
    n = x.shape[0]
    d = x.shape[1]
    h = w_on.shape[1]
    grid = (n // tile,)
    return pl.pallas_call(
        _node_mm_kernel,
        grid=grid,
        in_specs=[
            pl.BlockSpec((tile, d), lambda i: (i, 0)),
            pl.BlockSpec((d, h), lambda i: (0, 0)),
            pl.BlockSpec((1, h), lambda i: (0, 0)),
            pl.BlockSpec((d, h), lambda i: (0, 0)),
            pl.BlockSpec((1, h), lambda i: (0, 0)),
        ],
        out_specs=[
            pl.BlockSpec((tile, h), lambda i: (i, 0)),
            pl.BlockSpec((tile, h), lambda i: (i, 0)),
        ],
        out_shape=[
            jax.ShapeDtypeStruct((n, h), jnp.float32),
            jax.ShapeDtypeStruct((n, h), jnp.float32),
        ],
    )(x, w_on, b_on[None, :], w_off, b_off[None, :])


def _p1q1r1_kernel(u_ref, v_ref, invdeg_ref, w2on_ref, w2off_ref,
                   p1_ref, q1_ref, r1_ref):
    p1 = jax.nn.relu(u_ref[...] * invdeg_ref[...] + v_ref[...])
    p1_ref[...] = p1
    q1_ref[...] = p1 @ w2on_ref[...]
    r1_ref[...] = p1 @ w2off_ref[...]


def _p1q1r1(u, v, invdeg, w2_on, w2_off, tile=512):
    n, h = u.shape
    grid = (n // tile,)
    return pl.pallas_call(
        _p1q1r1_kernel,
        grid=grid,
        in_specs=[
            pl.BlockSpec((tile, h), lambda i: (i, 0)),
            pl.BlockSpec((tile, h), lambda i: (i, 0)),
            pl.BlockSpec((tile, 1), lambda i: (i, 0)),
            pl.BlockSpec((h, h), lambda i: (0, 0)),
            pl.BlockSpec((h, h), lambda i: (0, 0)),
        ],
        out_specs=[
            pl.BlockSpec((tile, h), lambda i: (i, 0)),
            pl.BlockSpec((tile, h), lambda i: (i, 0)),
            pl.BlockSpec((tile, h), lambda i: (i, 0)),
        ],
        out_shape=[
            jax.ShapeDtypeStruct((n, h), jnp.float32),
            jax.ShapeDtypeStruct((n, h), jnp.float32),
            jax.ShapeDtypeStruct((n, h), jnp.float32),
        ],
    )(u, v, invdeg[:, None], w2_on, w2_off)


def _edge_mm_kernel(g2_ref, w3on_ref, b3on_ref, w3off_ref, a3_ref, b3_ref):
    g2 = g2_ref[...]
    a3_ref[...] = g2 @ w3off_ref[...]
    b3_ref[...] = g2 @ w3on_ref[...] + b3on_ref[...]


def _edge_mm(g2, w3_on, b3_on, w3_off, tile=1024):
    e, h = g2.shape
    grid = (e // tile,)
    return pl.pallas_call(
        _edge_mm_kernel,
        grid=grid,
        in_specs=[
            pl.BlockSpec((tile, h), lambda i: (i, 0)),
            pl.BlockSpec((h, h), lambda i: (0, 0)),
            pl.BlockSpec((1, h), lambda i: (0, 0)),
            pl.BlockSpec((h, h), lambda i: (0, 0)),
        ],
        out_specs=[
            pl.BlockSpec((tile, h), lambda i: (i, 0)),
            pl.BlockSpec((tile, h), lambda i: (i, 0)),
        ],
        out_shape=[
            jax.ShapeDtypeStruct((e, h), jnp.float32),
            jax.ShapeDtypeStruct((e, h), jnp.float32),
        ],
    )(g2, w3_on, b3_on[None, :], w3_off)


def kernel(x, edge_index, batch, w1_on, b1_on, w1_off, b1_off, w2_on, b2_on,
           w2_off, b2_off, w3_on, b3_on, w3_off, b3_off, wl1, bl1, wl2, bl2):
    n = x.shape[0]
    src = edge_index[0]
    dst = edge_index[1]
    idx = edge_index.reshape(-1)

    deg = jax.ops.segment_sum(jnp.ones_like(idx, jnp.float32), idx,
                              num_segments=n)
    invdeg = 1.0 / jnp.maximum(deg, 1.0)

    U, V = _dual_matmul(x, w1_on, b1_on, w1_off, b1_off)
    P1, Q1, R1 = _p1q1r1(U, V, invdeg, w2_on, w2_off)

    acc = jax.ops.segment_sum(R1[dst], src, num_segments=n) + \
          jax.ops.segment_sum(R1[src], dst, num_segments=n)
    z2 = deg[:, None] * b2_off + (deg[:, None] * R1 + acc) * 0.5

    m = (Q1[src] + Q1[dst]) * 0.5 + b2_on
    u0 = (m + z2[src]) * invdeg[src][:, None]
    u1 = (m + z2[dst]) * invdeg[dst][:, None]
    g2 = (jax.nn.relu(u0) + jax.nn.relu(u1)) * 0.5

    A3, B3 = _edge_mm(g2, w3_on, b3_on, w3_off)
    z3 = jax.ops.segment_sum(A3, src, num_segments=n) + \
         jax.ops.segment_sum(A3, dst, num_segments=n) + deg[:, None] * b3_off

    v0 = (B3 + z3[src]) * invdeg[src][:, None]
    v1 = (B3 + z3[dst]) * invdeg[dst][:, None]
    g3 = (jax.nn.relu(v0) + jax.nn.relu(v1)) * 0.5

    w = (invdeg[src] + invdeg[dst])[:, None]
    total = jnp.sum(g3 * w, axis=0)
    g = (total / n)[None, :]

    h1 = jax.nn.relu(g @ wl1 + bl1)
    logits = h1 @ wl2 + bl2
    return jax.nn.log_softmax(logits, axis=-1)


# speedup vs baseline: 2.3983x; 1.3233x over previous
import functools
import jax
import jax.numpy as jnp
from jax import lax
from jax.experimental import pallas as pl
from jax.experimental.pallas import tpu as pltpu
from jax.experimental.pallas import tpu_sc as plsc

N = 10000
E = 320000
H = 128
NC = 2
NS = 16
NW = NC * NS
K = 80
NCHUNK = (E // NW) // K
NACC = 10240
SLICE = NACC // NS

_SC_MESH = plsc.VectorSubcoreMesh(
    core_axis_name="c", subcore_axis_name="s", num_cores=NC, num_subcores=NS)


def _wid():
    return lax.axis_index("s") * NC + lax.axis_index("c")


def _fill_value(ref, rows, value):
    vals = jnp.full((16,), value, jnp.float32)
    def body(i, _):
        r = i // 8
        h = i % 8
        ref[r, pl.ds(h * 16, 16)] = vals
        return 0
    lax.fori_loop(0, rows * 8, body, 0)


def _zero_acc_slice(acc_sh, zbuf):
    sid = lax.axis_index("s")
    zrows = zbuf.shape[0]
    _fill_value(zbuf, zrows, 0.0)
    base = pl.multiple_of(sid * SLICE, 8)
    for j in range(SLICE // zrows):
        pltpu.sync_copy(zbuf, acc_sh.at[pl.ds(base + j * zrows, zrows)])


def _dump_acc_slice(acc_sh, dbuf, out_hbm):
    cid = lax.axis_index("c")
    sid = lax.axis_index("s")
    drows = dbuf.shape[0]
    base = pl.multiple_of(sid * SLICE, 8)
    obase = pl.multiple_of(cid * NACC + base, 8)
    for j in range(SLICE // drows):
        pltpu.sync_copy(acc_sh.at[pl.ds(base + j * drows, drows)], dbuf)
        pltpu.sync_copy(dbuf, out_hbm.at[pl.ds(obase + j * drows, drows)])


def _make_deg_kernel():
    @functools.partial(
        pl.kernel,
        out_type=jax.ShapeDtypeStruct((NC * NACC, 128), jnp.float32),
        mesh=_SC_MESH,
        scratch_types=[
            pltpu.VMEM((2, K), jnp.int32),
            pltpu.VMEM((K, 128), jnp.float32),
            pltpu.VMEM((64, 128), jnp.float32),
            pltpu.VMEM_SHARED((NACC, 128), jnp.float32),
        ],
    )
    def deg_kernel(ei_hbm, degp_hbm, idx_v, ones_v, zbuf, acc_sh):
        w = _wid()
        _zero_acc_slice(acc_sh, zbuf)
        _fill_value(ones_v, K, 1.0)
        plsc.subcore_barrier()

        def chunk(c, _):
            ebase = pl.multiple_of(w * (E // NW) + c * K, 8)
            pltpu.sync_copy(ei_hbm.at[pl.ds(ebase, K)], idx_v.at[0])
            pltpu.sync_copy(ei_hbm.at[pl.ds(E + ebase, K)], idx_v.at[1])
            pltpu.sync_copy(ones_v, acc_sh.at[idx_v.at[0]], add=True)
            pltpu.sync_copy(ones_v, acc_sh.at[idx_v.at[1]], add=True)
            return 0
        lax.fori_loop(0, NCHUNK, chunk, 0)

        plsc.subcore_barrier()
        _dump_acc_slice(acc_sh, zbuf, degp_hbm)

    return deg_kernel


def _make_nbr_kernel():
    @functools.partial(
        pl.kernel,
        out_type=jax.ShapeDtypeStruct((NC * NACC, 128), jnp.float32),
        mesh=_SC_MESH,
        scratch_types=[
            pltpu.VMEM((2, K), jnp.int32),
            pltpu.VMEM((K, 128), jnp.float32),
            pltpu.VMEM((K, 128), jnp.float32),
            pltpu.VMEM((64, 128), jnp.float32),
            pltpu.VMEM_SHARED((NACC, 128), jnp.float32),
            pltpu.SemaphoreType.DMA,
        ],
    )
    def nbr_kernel(ei_hbm, r1_hbm, accp_hbm, idx_v, rows0, rows1, zbuf,
                   acc_sh, sem):
        w = _wid()
        _zero_acc_slice(acc_sh, zbuf)
        plsc.subcore_barrier()

        def chunk(c, _):
            ebase = pl.multiple_of(w * (E // NW) + c * K, 8)
            pltpu.sync_copy(ei_hbm.at[pl.ds(ebase, K)], idx_v.at[0])
            pltpu.sync_copy(ei_hbm.at[pl.ds(E + ebase, K)], idx_v.at[1])
            cp0 = pltpu.async_copy(r1_hbm.at[idx_v.at[1]], rows0, sem)
            cp1 = pltpu.async_copy(r1_hbm.at[idx_v.at[0]], rows1, sem)
            cp0.wait()
            pltpu.sync_copy(rows0, acc_sh.at[idx_v.at[0]], add=True)
            cp1.wait()
            pltpu.sync_copy(rows1, acc_sh.at[idx_v.at[1]], add=True)
            return 0
        lax.fori_loop(0, NCHUNK, chunk, 0)

        plsc.subcore_barrier()
        _dump_acc_slice(acc_sh, zbuf, accp_hbm)

    return nbr_kernel


def _make_a3scat_kernel():
    @functools.partial(
        pl.kernel,
        out_type=jax.ShapeDtypeStruct((NC * NACC, 128), jnp.float32),
        mesh=_SC_MESH,
        scratch_types=[
            pltpu.VMEM((2, K), jnp.int32),
            pltpu.VMEM((K, 128), jnp.float32),
            pltpu.VMEM((64, 128), jnp.float32),
            pltpu.VMEM_SHARED((NACC, 128), jnp.float32),
        ],
    )
    def a3scat_kernel(ei_hbm, a3_hbm, zp_hbm, idx_v, rows, zbuf, acc_sh):
        w = _wid()
        _zero_acc_slice(acc_sh, zbuf)
        plsc.subcore_barrier()

        def chunk(c, _):
            ebase = pl.multiple_of(w * (E // NW) + c * K, 8)
            pltpu.sync_copy(ei_hbm.at[pl.ds(ebase, K)], idx_v.at[0])
            pltpu.sync_copy(ei_hbm.at[pl.ds(E + ebase, K)], idx_v.at[1])
            pltpu.sync_copy(a3_hbm.at[pl.ds(ebase, K)], rows)
            pltpu.sync_copy(rows, acc_sh.at[idx_v.at[0]], add=True)
            pltpu.sync_copy(rows, acc_sh.at[idx_v.at[1]], add=True)
            return 0
        lax.fori_loop(0, NCHUNK, chunk, 0)

        plsc.subcore_barrier()
        _dump_acc_slice(acc_sh, zbuf, zp_hbm)

    return a3scat_kernel


def _node_mm_kernel(x_ref, won_ref, bon_ref, woff_ref, boff_ref, u_ref, v_ref):
    x = x_ref[...]
    u_ref[...] = x @ won_ref[...] + bon_ref[...]
    v_ref[...] = x @ woff_ref[...] + boff_ref[...]


def _dual_matmul(x, w_on, b_on, w_off, b_off, tile=1000):
    n = x.shape[0]
    d = x.shape[1]
    h = w_on.shape[1]
    grid = (n // tile,)
    return pl.pallas_call(
        _node_mm_kernel,
        grid=grid,
        in_specs=[
            pl.BlockSpec((tile, d), lambda i: (i, 0)),
            pl.BlockSpec((d, h), lambda i: (0, 0)),
            pl.BlockSpec((1, h), lambda i: (0, 0)),
            pl.BlockSpec((d, h), lambda i: (0, 0)),
            pl.BlockSpec((1, h), lambda i: (0, 0)),
        ],
        out_specs=[
            pl.BlockSpec((tile, h), lambda i: (i, 0)),
            pl.BlockSpec((tile, h), lambda i: (i, 0)),
        ],
        out_shape=[
            jax.ShapeDtypeStruct((n, h), jnp.float32),
            jax.ShapeDtypeStruct((n, h), jnp.float32),
        ],
    )(x, w_on, b_on[None, :], w_off, b_off[None, :])


def _p1q1r1_kernel(u_ref, v_ref, invdeg_ref, w2on_ref, w2off_ref,
                   p1_ref, q1_ref, r1_ref):
    p1 = jax.nn.relu(u_ref[...] * invdeg_ref[...] + v_ref[...])
    p1_ref[...] = p1
    q1_ref[...] = p1 @ w2on_ref[...]
    r1_ref[...] = p1 @ w2off_ref[...]


def _p1q1r1(u, v, invdeg, w2_on, w2_off, tile=1000):
    n, h = u.shape
    grid = (n // tile,)
    return pl.pallas_call(
        _p1q1r1_kernel,
        grid=grid,
        in_specs=[
            pl.BlockSpec((tile, h), lambda i: (i, 0)),
            pl.BlockSpec((tile, h), lambda i: (i, 0)),
            pl.BlockSpec((tile, 1), lambda i: (i, 0)),
            pl.BlockSpec((h, h), lambda i: (0, 0)),
            pl.BlockSpec((h, h), lambda i: (0, 0)),
        ],
        out_specs=[
            pl.BlockSpec((tile, h), lambda i: (i, 0)),
            pl.BlockSpec((tile, h), lambda i: (i, 0)),
            pl.BlockSpec((tile, h), lambda i: (i, 0)),
        ],
        out_shape=[
            jax.ShapeDtypeStruct((n, h), jnp.float32),
            jax.ShapeDtypeStruct((n, h), jnp.float32),
            jax.ShapeDtypeStruct((n, h), jnp.float32),
        ],
    )(u, v, invdeg[:, None], w2_on, w2_off)


def _edge_mm_kernel(g2_ref, w3on_ref, b3on_ref, w3off_ref, a3_ref, b3_ref):
    g2 = g2_ref[...]
    a3_ref[...] = g2 @ w3off_ref[...]
    b3_ref[...] = g2 @ w3on_ref[...] + b3on_ref[...]


def _edge_mm(g2, w3_on, b3_on, w3_off, tile=1280):
    e, h = g2.shape
    grid = (e // tile,)
    return pl.pallas_call(
        _edge_mm_kernel,
        grid=grid,
        in_specs=[
            pl.BlockSpec((tile, h), lambda i: (i, 0)),
            pl.BlockSpec((h, h), lambda i: (0, 0)),
            pl.BlockSpec((1, h), lambda i: (0, 0)),
            pl.BlockSpec((h, h), lambda i: (0, 0)),
        ],
        out_specs=[
            pl.BlockSpec((tile, h), lambda i: (i, 0)),
            pl.BlockSpec((tile, h), lambda i: (i, 0)),
        ],
        out_shape=[
            jax.ShapeDtypeStruct((e, h), jnp.float32),
            jax.ShapeDtypeStruct((e, h), jnp.float32),
        ],
    )(g2, w3_on, b3_on[None, :], w3_off)


def kernel(x, edge_index, batch, w1_on, b1_on, w1_off, b1_off, w2_on, b2_on,
           w2_off, b2_off, w3_on, b3_on, w3_off, b3_off, wl1, bl1, wl2, bl2):
    n = x.shape[0]
    src = edge_index[0]
    dst = edge_index[1]

    ei_flat = edge_index.reshape(-1)
    degp = _make_deg_kernel()(ei_flat)
    deg = degp[:N, 0] + degp[NACC:NACC + N, 0]
    invdeg = 1.0 / jnp.maximum(deg, 1.0)

    U, V = _dual_matmul(x, w1_on, b1_on, w1_off, b1_off)
    P1, Q1, R1 = _p1q1r1(U, V, invdeg, w2_on, w2_off)

    accp = _make_nbr_kernel()(ei_flat, R1)
    acc = accp[:N] + accp[NACC:NACC + N]
    z2 = deg[:, None] * b2_off + (deg[:, None] * R1 + acc) * 0.5

    m = (Q1[src] + Q1[dst]) * 0.5 + b2_on
    u0 = (m + z2[src]) * invdeg[src][:, None]
    u1 = (m + z2[dst]) * invdeg[dst][:, None]
    g2 = (jax.nn.relu(u0) + jax.nn.relu(u1)) * 0.5

    A3, B3 = _edge_mm(g2, w3_on, b3_on, w3_off)
    z3p = _make_a3scat_kernel()(ei_flat, A3)
    z3 = z3p[:N] + z3p[NACC:NACC + N] + deg[:, None] * b3_off

    v0 = (B3 + z3[src]) * invdeg[src][:, None]
    v1 = (B3 + z3[dst]) * invdeg[dst][:, None]
    g3 = (jax.nn.relu(v0) + jax.nn.relu(v1)) * 0.5

    w = (invdeg[src] + invdeg[dst])[:, None]
    total = jnp.sum(g3 * w, axis=0)
    g = (total / n)[None, :]

    h1 = jax.nn.relu(g @ wl1 + bl1)
    logits = h1 @ wl2 + bl2
    return jax.nn.log_softmax(logits, axis=-1)


# speedup vs baseline: 3.5354x; 1.4741x over previous
import functools
import jax
import jax.numpy as jnp
from jax import lax
from jax.experimental import pallas as pl
from jax.experimental.pallas import tpu as pltpu
from jax.experimental.pallas import tpu_sc as plsc

N = 10000
E = 320000
H = 128
NC = 2
NS = 16
NW = NC * NS
K = 80
NCHUNK = (E // NW) // K
NACC = 10240
SLICE = NACC // NS

_SC_MESH = plsc.VectorSubcoreMesh(
    core_axis_name="c", subcore_axis_name="s", num_cores=NC, num_subcores=NS)


def _wid():
    return lax.axis_index("s") * NC + lax.axis_index("c")


def _fill_value(ref, rows, value):
    vals = jnp.full((16,), value, jnp.float32)
    def body(i, _):
        r = i // 8
        h = i % 8
        ref[r, pl.ds(h * 16, 16)] = vals
        return 0
    lax.fori_loop(0, rows * 8, body, 0)


def _zero_acc_slice(acc_sh, zbuf):
    sid = lax.axis_index("s")
    zrows = zbuf.shape[0]
    _fill_value(zbuf, zrows, 0.0)
    base = pl.multiple_of(sid * SLICE, 8)
    for j in range(SLICE // zrows):
        pltpu.sync_copy(zbuf, acc_sh.at[pl.ds(base + j * zrows, zrows)])


def _dump_acc_slice(acc_sh, dbuf, out_hbm):
    cid = lax.axis_index("c")
    sid = lax.axis_index("s")
    drows = dbuf.shape[0]
    base = pl.multiple_of(sid * SLICE, 8)
    obase = pl.multiple_of(cid * NACC + base, 8)
    for j in range(SLICE // drows):
        pltpu.sync_copy(acc_sh.at[pl.ds(base + j * drows, drows)], dbuf)
        pltpu.sync_copy(dbuf, out_hbm.at[pl.ds(obase + j * drows, drows)])


def _make_deg_kernel():
    @functools.partial(
        pl.kernel,
        out_type=jax.ShapeDtypeStruct((NC * NACC, 128), jnp.float32),
        mesh=_SC_MESH,
        scratch_types=[
            pltpu.VMEM((2, K), jnp.int32),
            pltpu.VMEM((K, 128), jnp.float32),
            pltpu.VMEM((64, 128), jnp.float32),
            pltpu.VMEM_SHARED((NACC, 128), jnp.float32),
        ],
    )
    def deg_kernel(ei_hbm, degp_hbm, idx_v, ones_v, zbuf, acc_sh):
        w = _wid()
        _zero_acc_slice(acc_sh, zbuf)
        _fill_value(ones_v, K, 1.0)
        plsc.subcore_barrier()

        def chunk(c, _):
            ebase = pl.multiple_of(w * (E // NW) + c * K, 8)
            pltpu.sync_copy(ei_hbm.at[pl.ds(ebase, K)], idx_v.at[0])
            pltpu.sync_copy(ei_hbm.at[pl.ds(E + ebase, K)], idx_v.at[1])
            pltpu.sync_copy(ones_v, acc_sh.at[idx_v.at[0]], add=True)
            pltpu.sync_copy(ones_v, acc_sh.at[idx_v.at[1]], add=True)
            return 0
        lax.fori_loop(0, NCHUNK, chunk, 0)

        plsc.subcore_barrier()
        _dump_acc_slice(acc_sh, zbuf, degp_hbm)

    return deg_kernel


def _make_nbr_kernel():
    @functools.partial(
        pl.kernel,
        out_type=jax.ShapeDtypeStruct((NC * NACC, 128), jnp.float32),
        mesh=_SC_MESH,
        scratch_types=[
            pltpu.VMEM((2, K), jnp.int32),
            pltpu.VMEM((K, 128), jnp.float32),
            pltpu.VMEM((K, 128), jnp.float32),
            pltpu.VMEM((64, 128), jnp.float32),
            pltpu.VMEM_SHARED((NACC, 128), jnp.float32),
            pltpu.SemaphoreType.DMA,
        ],
    )
    def nbr_kernel(ei_hbm, r1_hbm, accp_hbm, idx_v, rows0, rows1, zbuf,
                   acc_sh, sem):
        w = _wid()
        _zero_acc_slice(acc_sh, zbuf)
        plsc.subcore_barrier()

        def chunk(c, _):
            ebase = pl.multiple_of(w * (E // NW) + c * K, 8)
            pltpu.sync_copy(ei_hbm.at[pl.ds(ebase, K)], idx_v.at[0])
            pltpu.sync_copy(ei_hbm.at[pl.ds(E + ebase, K)], idx_v.at[1])
            cp0 = pltpu.async_copy(r1_hbm.at[idx_v.at[1]], rows0, sem)
            cp1 = pltpu.async_copy(r1_hbm.at[idx_v.at[0]], rows1, sem)
            cp0.wait()
            pltpu.sync_copy(rows0, acc_sh.at[idx_v.at[0]], add=True)
            cp1.wait()
            pltpu.sync_copy(rows1, acc_sh.at[idx_v.at[1]], add=True)
            return 0
        lax.fori_loop(0, NCHUNK, chunk, 0)

        plsc.subcore_barrier()
        _dump_acc_slice(acc_sh, zbuf, accp_hbm)

    return nbr_kernel


def _make_a3scat_kernel():
    @functools.partial(
        pl.kernel,
        out_type=jax.ShapeDtypeStruct((NC * NACC, 128), jnp.float32),
        mesh=_SC_MESH,
        scratch_types=[
            pltpu.VMEM((2, K), jnp.int32),
            pltpu.VMEM((K, 128), jnp.float32),
            pltpu.VMEM((64, 128), jnp.float32),
            pltpu.VMEM_SHARED((NACC, 128), jnp.float32),
        ],
    )
    def a3scat_kernel(ei_hbm, a3_hbm, zp_hbm, idx_v, rows, zbuf, acc_sh):
        w = _wid()
        _zero_acc_slice(acc_sh, zbuf)
        plsc.subcore_barrier()

        def chunk(c, _):
            ebase = pl.multiple_of(w * (E // NW) + c * K, 8)
            pltpu.sync_copy(ei_hbm.at[pl.ds(ebase, K)], idx_v.at[0])
            pltpu.sync_copy(ei_hbm.at[pl.ds(E + ebase, K)], idx_v.at[1])
            pltpu.sync_copy(a3_hbm.at[pl.ds(ebase, K)], rows)
            pltpu.sync_copy(rows, acc_sh.at[idx_v.at[0]], add=True)
            pltpu.sync_copy(rows, acc_sh.at[idx_v.at[1]], add=True)
            return 0
        lax.fori_loop(0, NCHUNK, chunk, 0)

        plsc.subcore_barrier()
        _dump_acc_slice(acc_sh, zbuf, zp_hbm)

    return a3scat_kernel


def _make_gath2_kernel():
    @functools.partial(
        pl.kernel,
        out_type=[jax.ShapeDtypeStruct((E, 128), jnp.float32),
                  jax.ShapeDtypeStruct((E, 128), jnp.float32)],
        mesh=_SC_MESH,
        scratch_types=[
            pltpu.VMEM((2, K), jnp.int32),
            pltpu.VMEM((K, 128), jnp.float32),
            pltpu.VMEM((K, 128), jnp.float32),
            pltpu.SemaphoreType.DMA,
        ],
    )
    def gath2_kernel(ei_hbm, t_hbm, outs_hbm, outd_hbm, idx_v, buf0, buf1,
                     sem):
        w = _wid()

        def chunk(c, _):
            ebase = pl.multiple_of(w * (E // NW) + c * K, 8)
            pltpu.sync_copy(ei_hbm.at[pl.ds(ebase, K)], idx_v.at[0])
            pltpu.sync_copy(ei_hbm.at[pl.ds(E + ebase, K)], idx_v.at[1])
            cp0 = pltpu.async_copy(t_hbm.at[idx_v.at[0]], buf0, sem)
            cp1 = pltpu.async_copy(t_hbm.at[idx_v.at[1]], buf1, sem)
            cp0.wait()
            pltpu.sync_copy(buf0, outs_hbm.at[pl.ds(ebase, K)])
            cp1.wait()
            pltpu.sync_copy(buf1, outd_hbm.at[pl.ds(ebase, K)])
            return 0
        lax.fori_loop(0, NCHUNK, chunk, 0)

    return gath2_kernel


def _make_gath4_kernel():
    @functools.partial(
        pl.kernel,
        out_type=[jax.ShapeDtypeStruct((E, 128), jnp.float32),
                  jax.ShapeDtypeStruct((E, 128), jnp.float32),
                  jax.ShapeDtypeStruct((E, 128), jnp.float32),
                  jax.ShapeDtypeStruct((E, 128), jnp.float32)],
        mesh=_SC_MESH,
        scratch_types=[
            pltpu.VMEM((2, K), jnp.int32),
            pltpu.VMEM((K, 128), jnp.float32),
            pltpu.VMEM((K, 128), jnp.float32),
            pltpu.SemaphoreType.DMA,
        ],
    )
    def gath4_kernel(ei_hbm, ta_hbm, tb_hbm, as_hbm, ad_hbm, bs_hbm, bd_hbm,
                     idx_v, buf0, buf1, sem):
        w = _wid()

        def chunk(c, _):
            ebase = pl.multiple_of(w * (E // NW) + c * K, 8)
            pltpu.sync_copy(ei_hbm.at[pl.ds(ebase, K)], idx_v.at[0])
            pltpu.sync_copy(ei_hbm.at[pl.ds(E + ebase, K)], idx_v.at[1])
            cp0 = pltpu.async_copy(ta_hbm.at[idx_v.at[0]], buf0, sem)
            cp1 = pltpu.async_copy(ta_hbm.at[idx_v.at[1]], buf1, sem)
            cp0.wait()
            pltpu.sync_copy(buf0, as_hbm.at[pl.ds(ebase, K)])
            cp1.wait()
            pltpu.sync_copy(buf1, ad_hbm.at[pl.ds(ebase, K)])
            cp2 = pltpu.async_copy(tb_hbm.at[idx_v.at[0]], buf0, sem)
            cp3 = pltpu.async_copy(tb_hbm.at[idx_v.at[1]], buf1, sem)
            cp2.wait()
            pltpu.sync_copy(buf0, bs_hbm.at[pl.ds(ebase, K)])
            cp3.wait()
            pltpu.sync_copy(buf1, bd_hbm.at[pl.ds(ebase, K)])
            return 0
        lax.fori_loop(0, NCHUNK, chunk, 0)

    return gath4_kernel


def _node_mm_kernel(x_ref, won_ref, bon_ref, woff_ref, boff_ref, u_ref, v_ref):
    x = x_ref[...]
    u_ref[...] = x @ won_ref[...] + bon_ref[...]
    v_ref[...] = x @ woff_ref[...] + boff_ref[...]


def _dual_matmul(x, w_on, b_on, w_off, b_off, tile=1000):
    n = x.shape[0]
    d = x.shape[1]
    h = w_on.shape[1]
    grid = (n // tile,)
    return pl.pallas_call(
        _node_mm_kernel,
        grid=grid,
        in_specs=[
            pl.BlockSpec((tile, d), lambda i: (i, 0)),
            pl.BlockSpec((d, h), lambda i: (0, 0)),
            pl.BlockSpec((1, h), lambda i: (0, 0)),
            pl.BlockSpec((d, h), lambda i: (0, 0)),
            pl.BlockSpec((1, h), lambda i: (0, 0)),
        ],
        out_specs=[
            pl.BlockSpec((tile, h), lambda i: (i, 0)),
            pl.BlockSpec((tile, h), lambda i: (i, 0)),
        ],
        out_shape=[
            jax.ShapeDtypeStruct((n, h), jnp.float32),
            jax.ShapeDtypeStruct((n, h), jnp.float32),
        ],
    )(x, w_on, b_on[None, :], w_off, b_off[None, :])


def _p1q1r1_kernel(u_ref, v_ref, invdeg_ref, w2on_ref, w2off_ref,
                   q1h_ref, r1_ref):
    p1 = jax.nn.relu(u_ref[...] * invdeg_ref[...] + v_ref[...])
    q1h_ref[...] = (p1 @ w2on_ref[...]) * 0.5
    r1_ref[...] = p1 @ w2off_ref[...]


def _p1q1r1(u, v, invdeg, w2_on, w2_off, tile=1000):
    n, h = u.shape
    grid = (n // tile,)
    return pl.pallas_call(
        _p1q1r1_kernel,
        grid=grid,
        in_specs=[
            pl.BlockSpec((tile, h), lambda i: (i, 0)),
            pl.BlockSpec((tile, h), lambda i: (i, 0)),
            pl.BlockSpec((tile, 1), lambda i: (i, 0)),
            pl.BlockSpec((h, h), lambda i: (0, 0)),
            pl.BlockSpec((h, h), lambda i: (0, 0)),
        ],
        out_specs=[
            pl.BlockSpec((tile, h), lambda i: (i, 0)),
            pl.BlockSpec((tile, h), lambda i: (i, 0)),
        ],
        out_shape=[
            jax.ShapeDtypeStruct((n, h), jnp.float32),
            jax.ShapeDtypeStruct((n, h), jnp.float32),
        ],
    )(u, v, invdeg[:, None], w2_on, w2_off)


def _stage2_kernel(ts_ref, td_ref, z2s_ref, z2d_ref, ws_ref, wd_ref,
                   b2on_ref, w3on_ref, b3on_ref, w3off_ref, a3_ref, b3_ref):
    m = ts_ref[...] + td_ref[...] + b2on_ref[...]
    u0 = (m + z2s_ref[...]) * ws_ref[...]
    u1 = (m + z2d_ref[...]) * wd_ref[...]
    g2 = (jax.nn.relu(u0) + jax.nn.relu(u1)) * 0.5
    a3_ref[...] = g2 @ w3off_ref[...]
    b3_ref[...] = g2 @ w3on_ref[...] + b3on_ref[...]


def _stage2(ts, td, z2s, z2d, ws, wd, b2_on, w3_on, b3_on, w3_off, tile=1280):
    e, h = ts.shape
    grid = (e // tile,)
    row = lambda i: (i, 0)
    fixed = lambda i: (0, 0)
    return pl.pallas_call(
        _stage2_kernel,
        grid=grid,
        in_specs=[
            pl.BlockSpec((tile, h), row),
            pl.BlockSpec((tile, h), row),
            pl.BlockSpec((tile, h), row),
            pl.BlockSpec((tile, h), row),
            pl.BlockSpec((tile, 1), row),
            pl.BlockSpec((tile, 1), row),
            pl.BlockSpec((1, h), fixed),
            pl.BlockSpec((h, h), fixed),
            pl.BlockSpec((1, h), fixed),
            pl.BlockSpec((h, h), fixed),
        ],
        out_specs=[
            pl.BlockSpec((tile, h), row),
            pl.BlockSpec((tile, h), row),
        ],
        out_shape=[
            jax.ShapeDtypeStruct((e, h), jnp.float32),
            jax.ShapeDtypeStruct((e, h), jnp.float32),
        ],
    )(ts, td, z2s, z2d, ws[:, None], wd[:, None],
      b2_on[None, :], w3_on, b3_on[None, :], w3_off)


def _stage3_kernel(b3_ref, z3s_ref, z3d_ref, ws_ref, wd_ref, part_ref):
    i = pl.program_id(0)
    b3 = b3_ref[...]
    ws = ws_ref[...]
    wd = wd_ref[...]
    v0 = (b3 + z3s_ref[...]) * ws
    v1 = (b3 + z3d_ref[...]) * wd
    g3w = (jax.nn.relu(v0) + jax.nn.relu(v1)) * ((ws + wd) * 0.5)
    psum = jnp.sum(g3w.reshape(-1, 8, 128), axis=0)

    @pl.when(i == 0)
    def _():
        part_ref[...] = jnp.zeros_like(part_ref)
    part_ref[...] += psum


def _stage3(b3, z3s, z3d, ws, wd, tile=1280):
    e, h = b3.shape
    grid = (e // tile,)
    row = lambda i: (i, 0)
    return pl.pallas_call(
        _stage3_kernel,
        grid=grid,
        in_specs=[
            pl.BlockSpec((tile, h), row),
            pl.BlockSpec((tile, h), row),
            pl.BlockSpec((tile, h), row),
            pl.BlockSpec((tile, 1), row),
            pl.BlockSpec((tile, 1), row),
        ],
        out_specs=pl.BlockSpec((8, h), lambda i: (0, 0)),
        out_shape=jax.ShapeDtypeStruct((8, h), jnp.float32),
    )(b3, z3s, z3d, ws[:, None], wd[:, None])


def kernel(x, edge_index, batch, w1_on, b1_on, w1_off, b1_off, w2_on, b2_on,
           w2_off, b2_off, w3_on, b3_on, w3_off, b3_off, wl1, bl1, wl2, bl2):
    n = x.shape[0]
    src = edge_index[0]
    dst = edge_index[1]

    ei_flat = edge_index.reshape(-1)
    degp = _make_deg_kernel()(ei_flat)
    deg = degp[:N, 0] + degp[NACC:NACC + N, 0]
    invdeg = 1.0 / jnp.maximum(deg, 1.0)
    ws = invdeg[src]
    wd = invdeg[dst]

    U, V = _dual_matmul(x, w1_on, b1_on, w1_off, b1_off)
    Q1h, R1 = _p1q1r1(U, V, invdeg, w2_on, w2_off)

    accp = _make_nbr_kernel()(ei_flat, R1)
    acc = accp[:N] + accp[NACC:NACC + N]
    z2 = deg[:, None] * b2_off + (deg[:, None] * R1 + acc) * 0.5

    Ts, Td, z2s, z2d = _make_gath4_kernel()(ei_flat, Q1h, z2)
    A3, B3 = _stage2(Ts, Td, z2s, z2d, ws, wd, b2_on, w3_on, b3_on, w3_off)

    z3p = _make_a3scat_kernel()(ei_flat, A3)
    z3 = z3p[:N] + z3p[NACC:NACC + N] + deg[:, None] * b3_off
    z3s, z3d = _make_gath2_kernel()(ei_flat, z3)

    parts = _stage3(B3, z3s, z3d, ws, wd)
    total = jnp.sum(parts, axis=0)
    g = (total / n)[None, :]

    h1 = jax.nn.relu(g @ wl1 + bl1)
    logits = h1 @ wl2 + bl2
    return jax.nn.log_softmax(logits, axis=-1)


# speedup vs baseline: 7.2196x; 2.0421x over previous
import functools
import jax
import jax.numpy as jnp
from jax import lax
from jax.experimental import pallas as pl
from jax.experimental.pallas import tpu as pltpu
from jax.experimental.pallas import tpu_sc as plsc

N = 10000
E = 320000
H = 128
NC = 2
NS = 16
NW = NC * NS
K = 80
NCHUNK = (E // NW) // K
NACC = 10240
SLICE = NACC // NS

_SC_MESH = plsc.VectorSubcoreMesh(
    core_axis_name="c", subcore_axis_name="s", num_cores=NC, num_subcores=NS)


def _wid():
    return lax.axis_index("s") * NC + lax.axis_index("c")


def _fill_value(ref, rows, value):
    vals = jnp.full((16,), value, jnp.float32)
    def body(i, _):
        r = i // 8
        h = i % 8
        ref[r, pl.ds(h * 16, 16)] = vals
        return 0
    lax.fori_loop(0, rows * 8, body, 0)


def _zero_acc_slice(acc_sh, zbuf):
    sid = lax.axis_index("s")
    zrows = zbuf.shape[0]
    _fill_value(zbuf, zrows, 0.0)
    base = pl.multiple_of(sid * SLICE, 8)
    for j in range(SLICE // zrows):
        pltpu.sync_copy(zbuf, acc_sh.at[pl.ds(base + j * zrows, zrows)])


def _dump_acc_slice(acc_sh, dbuf, out_hbm):
    cid = lax.axis_index("c")
    sid = lax.axis_index("s")
    drows = dbuf.shape[0]
    base = pl.multiple_of(sid * SLICE, 8)
    obase = pl.multiple_of(cid * NACC + base, 8)
    for j in range(SLICE // drows):
        pltpu.sync_copy(acc_sh.at[pl.ds(base + j * drows, drows)], dbuf)
        pltpu.sync_copy(dbuf, out_hbm.at[pl.ds(obase + j * drows, drows)])


def _make_deg_kernel():
    @functools.partial(
        pl.kernel,
        out_type=jax.ShapeDtypeStruct((NC * NACC, 128), jnp.float32),
        mesh=_SC_MESH,
        scratch_types=[
            pltpu.VMEM((2, K), jnp.int32),
            pltpu.VMEM((K, 128), jnp.float32),
            pltpu.VMEM((64, 128), jnp.float32),
            pltpu.VMEM_SHARED((NACC, 128), jnp.float32),
        ],
    )
    def deg_kernel(ei_hbm, degp_hbm, idx_v, ones_v, zbuf, acc_sh):
        w = _wid()
        _zero_acc_slice(acc_sh, zbuf)
        _fill_value(ones_v, K, 1.0)
        plsc.subcore_barrier()

        def chunk(c, _):
            ebase = pl.multiple_of(w * (E // NW) + c * K, 8)
            pltpu.sync_copy(ei_hbm.at[pl.ds(ebase, K)], idx_v.at[0])
            pltpu.sync_copy(ei_hbm.at[pl.ds(E + ebase, K)], idx_v.at[1])
            pltpu.sync_copy(ones_v, acc_sh.at[idx_v.at[0]], add=True)
            pltpu.sync_copy(ones_v, acc_sh.at[idx_v.at[1]], add=True)
            return 0
        lax.fori_loop(0, NCHUNK, chunk, 0)

        plsc.subcore_barrier()
        _dump_acc_slice(acc_sh, zbuf, degp_hbm)

    return deg_kernel


def _make_nbr_kernel():
    @functools.partial(
        pl.kernel,
        out_type=jax.ShapeDtypeStruct((NC * NACC, 128), jnp.float32),
        mesh=_SC_MESH,
        scratch_types=[
            pltpu.VMEM((2, K), jnp.int32),
            pltpu.VMEM((K, 128), jnp.float32),
            pltpu.VMEM((K, 128), jnp.float32),
            pltpu.VMEM((64, 128), jnp.float32),
            pltpu.VMEM_SHARED((NACC, 128), jnp.float32),
            pltpu.SemaphoreType.DMA,
        ],
    )
    def nbr_kernel(ei_hbm, r1_hbm, accp_hbm, idx_v, rows0, rows1, zbuf,
                   acc_sh, sem):
        w = _wid()
        _zero_acc_slice(acc_sh, zbuf)
        plsc.subcore_barrier()

        def chunk(c, _):
            ebase = pl.multiple_of(w * (E // NW) + c * K, 8)
            pltpu.sync_copy(ei_hbm.at[pl.ds(ebase, K)], idx_v.at[0])
            pltpu.sync_copy(ei_hbm.at[pl.ds(E + ebase, K)], idx_v.at[1])
            cp0 = pltpu.async_copy(r1_hbm.at[idx_v.at[1]], rows0, sem)
            cp1 = pltpu.async_copy(r1_hbm.at[idx_v.at[0]], rows1, sem)
            cp0.wait()
            pltpu.sync_copy(rows0, acc_sh.at[idx_v.at[0]], add=True)
            cp1.wait()
            pltpu.sync_copy(rows1, acc_sh.at[idx_v.at[1]], add=True)
            return 0
        lax.fori_loop(0, NCHUNK, chunk, 0)

        plsc.subcore_barrier()
        _dump_acc_slice(acc_sh, zbuf, accp_hbm)

    return nbr_kernel


def _make_a3scat_kernel():
    @functools.partial(
        pl.kernel,
        out_type=jax.ShapeDtypeStruct((NC * NACC, 128), jnp.float32),
        mesh=_SC_MESH,
        scratch_types=[
            pltpu.VMEM((2, K), jnp.int32),
            pltpu.VMEM((K, 128), jnp.float32),
            pltpu.VMEM((64, 128), jnp.float32),
            pltpu.VMEM_SHARED((NACC, 128), jnp.float32),
        ],
    )
    def a3scat_kernel(ei_hbm, a3_hbm, zp_hbm, idx_v, rows, zbuf, acc_sh):
        w = _wid()
        _zero_acc_slice(acc_sh, zbuf)
        plsc.subcore_barrier()

        def chunk(c, _):
            ebase = pl.multiple_of(w * (E // NW) + c * K, 8)
            pltpu.sync_copy(ei_hbm.at[pl.ds(ebase, K)], idx_v.at[0])
            pltpu.sync_copy(ei_hbm.at[pl.ds(E + ebase, K)], idx_v.at[1])
            pltpu.sync_copy(a3_hbm.at[pl.ds(ebase, K)], rows)
            pltpu.sync_copy(rows, acc_sh.at[idx_v.at[0]], add=True)
            pltpu.sync_copy(rows, acc_sh.at[idx_v.at[1]], add=True)
            return 0
        lax.fori_loop(0, NCHUNK, chunk, 0)

        plsc.subcore_barrier()
        _dump_acc_slice(acc_sh, zbuf, zp_hbm)

    return a3scat_kernel


def _make_gath2_kernel():
    @functools.partial(
        pl.kernel,
        out_type=[jax.ShapeDtypeStruct((E, 128), jnp.float32),
                  jax.ShapeDtypeStruct((E, 128), jnp.float32)],
        mesh=_SC_MESH,
        scratch_types=[
            pltpu.VMEM((2, K), jnp.int32),
            pltpu.VMEM((K, 128), jnp.float32),
            pltpu.VMEM((K, 128), jnp.float32),
            pltpu.SemaphoreType.DMA,
        ],
    )
    def gath2_kernel(ei_hbm, t_hbm, outs_hbm, outd_hbm, idx_v, buf0, buf1,
                     sem):
        w = _wid()

        def chunk(c, _):
            ebase = pl.multiple_of(w * (E // NW) + c * K, 8)
            pltpu.sync_copy(ei_hbm.at[pl.ds(ebase, K)], idx_v.at[0])
            pltpu.sync_copy(ei_hbm.at[pl.ds(E + ebase, K)], idx_v.at[1])
            cp0 = pltpu.async_copy(t_hbm.at[idx_v.at[0]], buf0, sem)
            cp1 = pltpu.async_copy(t_hbm.at[idx_v.at[1]], buf1, sem)
            cp0.wait()
            pltpu.sync_copy(buf0, outs_hbm.at[pl.ds(ebase, K)])
            cp1.wait()
            pltpu.sync_copy(buf1, outd_hbm.at[pl.ds(ebase, K)])
            return 0
        lax.fori_loop(0, NCHUNK, chunk, 0)

    return gath2_kernel


def _make_gath6_kernel():
    @functools.partial(
        pl.kernel,
        out_type=[jax.ShapeDtypeStruct((E, 128), jnp.float32)
                  for _ in range(6)],
        mesh=_SC_MESH,
        scratch_types=[
            pltpu.VMEM((2, K), jnp.int32),
            pltpu.VMEM((K, 128), jnp.float32),
            pltpu.VMEM((K, 128), jnp.float32),
            pltpu.SemaphoreType.DMA,
        ],
    )
    def gath6_kernel(ei_hbm, ta_hbm, tb_hbm, tc_hbm, as_hbm, ad_hbm, bs_hbm,
                     bd_hbm, cs_hbm, cd_hbm, idx_v, buf0, buf1, sem):
        w = _wid()

        def chunk(c, _):
            ebase = pl.multiple_of(w * (E // NW) + c * K, 8)
            pltpu.sync_copy(ei_hbm.at[pl.ds(ebase, K)], idx_v.at[0])
            pltpu.sync_copy(ei_hbm.at[pl.ds(E + ebase, K)], idx_v.at[1])
            for t_hbm, os_hbm, od_hbm in ((ta_hbm, as_hbm, ad_hbm),
                                          (tb_hbm, bs_hbm, bd_hbm),
                                          (tc_hbm, cs_hbm, cd_hbm)):
                cp0 = pltpu.async_copy(t_hbm.at[idx_v.at[0]], buf0, sem)
                cp1 = pltpu.async_copy(t_hbm.at[idx_v.at[1]], buf1, sem)
                cp0.wait()
                pltpu.sync_copy(buf0, os_hbm.at[pl.ds(ebase, K)])
                cp1.wait()
                pltpu.sync_copy(buf1, od_hbm.at[pl.ds(ebase, K)])
            return 0
        lax.fori_loop(0, NCHUNK, chunk, 0)

    return gath6_kernel


def _node_mm_kernel(x_ref, won_ref, bon_ref, woff_ref, boff_ref, u_ref, v_ref):
    x = x_ref[...]
    u_ref[...] = x @ won_ref[...] + bon_ref[...]
    v_ref[...] = x @ woff_ref[...] + boff_ref[...]


def _dual_matmul(x, w_on, b_on, w_off, b_off, tile=1000):
    n = x.shape[0]
    d = x.shape[1]
    h = w_on.shape[1]
    grid = (n // tile,)
    return pl.pallas_call(
        _node_mm_kernel,
        grid=grid,
        in_specs=[
            pl.BlockSpec((tile, d), lambda i: (i, 0)),
            pl.BlockSpec((d, h), lambda i: (0, 0)),
            pl.BlockSpec((1, h), lambda i: (0, 0)),
            pl.BlockSpec((d, h), lambda i: (0, 0)),
            pl.BlockSpec((1, h), lambda i: (0, 0)),
        ],
        out_specs=[
            pl.BlockSpec((tile, h), lambda i: (i, 0)),
            pl.BlockSpec((tile, h), lambda i: (i, 0)),
        ],
        out_shape=[
            jax.ShapeDtypeStruct((n, h), jnp.float32),
            jax.ShapeDtypeStruct((n, h), jnp.float32),
        ],
    )(x, w_on, b_on[None, :], w_off, b_off[None, :])


def _p1q1r1_kernel(u_ref, v_ref, invdeg_ref, w2on_ref, w2off_ref,
                   q1h_ref, r1_ref):
    p1 = jax.nn.relu(u_ref[...] * invdeg_ref[...] + v_ref[...])
    q1h_ref[...] = (p1 @ w2on_ref[...]) * 0.5
    r1_ref[...] = p1 @ w2off_ref[...]


def _p1q1r1(u, v, invdeg, w2_on, w2_off, tile=1000):
    n, h = u.shape
    grid = (n // tile,)
    return pl.pallas_call(
        _p1q1r1_kernel,
        grid=grid,
        in_specs=[
            pl.BlockSpec((tile, h), lambda i: (i, 0)),
            pl.BlockSpec((tile, h), lambda i: (i, 0)),
            pl.BlockSpec((tile, h), lambda i: (i, 0)),
            pl.BlockSpec((h, h), lambda i: (0, 0)),
            pl.BlockSpec((h, h), lambda i: (0, 0)),
        ],
        out_specs=[
            pl.BlockSpec((tile, h), lambda i: (i, 0)),
            pl.BlockSpec((tile, h), lambda i: (i, 0)),
        ],
        out_shape=[
            jax.ShapeDtypeStruct((n, h), jnp.float32),
            jax.ShapeDtypeStruct((n, h), jnp.float32),
        ],
    )(u, v, invdeg, w2_on, w2_off)


def _stage2_kernel(ts_ref, td_ref, z2s_ref, z2d_ref, ws_ref, wd_ref,
                   b2on_ref, w3on_ref, b3on_ref, w3off_ref, a3_ref, b3_ref):
    m = ts_ref[...] + td_ref[...] + b2on_ref[...]
    u0 = (m + z2s_ref[...]) * ws_ref[...]
    u1 = (m + z2d_ref[...]) * wd_ref[...]
    g2 = (jax.nn.relu(u0) + jax.nn.relu(u1)) * 0.5
    a3_ref[...] = g2 @ w3off_ref[...]
    b3_ref[...] = g2 @ w3on_ref[...] + b3on_ref[...]


def _stage2(ts, td, z2s, z2d, ws, wd, b2_on, w3_on, b3_on, w3_off, tile=1280):
    e, h = ts.shape
    grid = (e // tile,)
    row = lambda i: (i, 0)
    fixed = lambda i: (0, 0)
    return pl.pallas_call(
        _stage2_kernel,
        grid=grid,
        in_specs=[
            pl.BlockSpec((tile, h), row),
            pl.BlockSpec((tile, h), row),
            pl.BlockSpec((tile, h), row),
            pl.BlockSpec((tile, h), row),
            pl.BlockSpec((tile, h), row),
            pl.BlockSpec((tile, h), row),
            pl.BlockSpec((1, h), fixed),
            pl.BlockSpec((h, h), fixed),
            pl.BlockSpec((1, h), fixed),
            pl.BlockSpec((h, h), fixed),
        ],
        out_specs=[
            pl.BlockSpec((tile, h), row),
            pl.BlockSpec((tile, h), row),
        ],
        out_shape=[
            jax.ShapeDtypeStruct((e, h), jnp.float32),
            jax.ShapeDtypeStruct((e, h), jnp.float32),
        ],
    )(ts, td, z2s, z2d, ws, wd,
      b2_on[None, :], w3_on, b3_on[None, :], w3_off)


def _stage3_kernel(b3_ref, z3s_ref, z3d_ref, ws_ref, wd_ref, part_ref):
    i = pl.program_id(0)
    b3 = b3_ref[...]
    ws = ws_ref[...]
    wd = wd_ref[...]
    v0 = (b3 + z3s_ref[...]) * ws
    v1 = (b3 + z3d_ref[...]) * wd
    g3w = (jax.nn.relu(v0) + jax.nn.relu(v1)) * ((ws + wd) * 0.5)
    psum = jnp.sum(g3w.reshape(-1, 8, 128), axis=0)

    @pl.when(i == 0)
    def _():
        part_ref[...] = jnp.zeros_like(part_ref)
    part_ref[...] += psum


def _stage3(b3, z3s, z3d, ws, wd, tile=1280):
    e, h = b3.shape
    grid = (e // tile,)
    row = lambda i: (i, 0)
    return pl.pallas_call(
        _stage3_kernel,
        grid=grid,
        in_specs=[
            pl.BlockSpec((tile, h), row),
            pl.BlockSpec((tile, h), row),
            pl.BlockSpec((tile, h), row),
            pl.BlockSpec((tile, h), row),
            pl.BlockSpec((tile, h), row),
        ],
        out_specs=pl.BlockSpec((8, h), lambda i: (0, 0)),
        out_shape=jax.ShapeDtypeStruct((8, h), jnp.float32),
    )(b3, z3s, z3d, ws, wd)


def kernel(x, edge_index, batch, w1_on, b1_on, w1_off, b1_off, w2_on, b2_on,
           w2_off, b2_off, w3_on, b3_on, w3_off, b3_off, wl1, bl1, wl2, bl2):
    n = x.shape[0]

    ei_flat = edge_index.reshape(-1)
    degp = _make_deg_kernel()(ei_flat)
    deg = degp[:N] + degp[NACC:NACC + N]
    invdeg = 1.0 / jnp.maximum(deg, 1.0)

    U, V = _dual_matmul(x, w1_on, b1_on, w1_off, b1_off)
    Q1h, R1 = _p1q1r1(U, V, invdeg, w2_on, w2_off)

    accp = _make_nbr_kernel()(ei_flat, R1)
    acc = accp[:N] + accp[NACC:NACC + N]
    z2 = deg * b2_off + (deg * R1 + acc) * 0.5

    Ts, Td, z2s, z2d, ws, wd = _make_gath6_kernel()(ei_flat, Q1h, z2, invdeg)
    A3, B3 = _stage2(Ts, Td, z2s, z2d, ws, wd, b2_on, w3_on, b3_on, w3_off)

    z3p = _make_a3scat_kernel()(ei_flat, A3)
    z3 = z3p[:N] + z3p[NACC:NACC + N] + deg * b3_off
    z3s, z3d = _make_gath2_kernel()(ei_flat, z3)

    parts = _stage3(B3, z3s, z3d, ws, wd)
    total = jnp.sum(parts, axis=0)
    g = (total / n)[None, :]

    h1 = jax.nn.relu(g @ wl1 + bl1)
    logits = h1 @ wl2 + bl2
    return jax.nn.log_softmax(logits, axis=-1)


# speedup vs baseline: 7.2484x; 1.0040x over previous
import functools
import jax
import jax.numpy as jnp
from jax import lax
from jax.experimental import pallas as pl
from jax.experimental.pallas import tpu as pltpu
from jax.experimental.pallas import tpu_sc as plsc

N = 10000
E = 320000
H = 128
NC = 2
NS = 16
NW = NC * NS
K = 80
NCHUNK = (E // NW) // K
NACC = 10240
SLICE = NACC // NS

_SC_MESH = plsc.VectorSubcoreMesh(
    core_axis_name="c", subcore_axis_name="s", num_cores=NC, num_subcores=NS)


def _wid():
    return lax.axis_index("s") * NC + lax.axis_index("c")


def _fill_value(ref, rows, value):
    vals = jnp.full((16,), value, jnp.float32)
    def body(i, _):
        r = i // 8
        h = i % 8
        ref[r, pl.ds(h * 16, 16)] = vals
        return 0
    lax.fori_loop(0, rows * 8, body, 0)


def _zero_acc_slice(acc_sh, zbuf):
    sid = lax.axis_index("s")
    zrows = zbuf.shape[0]
    _fill_value(zbuf, zrows, 0.0)
    base = pl.multiple_of(sid * SLICE, 8)
    for j in range(SLICE // zrows):
        pltpu.sync_copy(zbuf, acc_sh.at[pl.ds(base + j * zrows, zrows)])


def _dump_acc_slice(acc_sh, dbuf, out_hbm):
    cid = lax.axis_index("c")
    sid = lax.axis_index("s")
    drows = dbuf.shape[0]
    base = pl.multiple_of(sid * SLICE, 8)
    obase = pl.multiple_of(cid * NACC + base, 8)
    for j in range(SLICE // drows):
        pltpu.sync_copy(acc_sh.at[pl.ds(base + j * drows, drows)], dbuf)
        pltpu.sync_copy(dbuf, out_hbm.at[pl.ds(obase + j * drows, drows)])


def _make_deg_kernel():
    @functools.partial(
        pl.kernel,
        out_type=jax.ShapeDtypeStruct((NC * NACC, 128), jnp.float32),
        mesh=_SC_MESH,
        scratch_types=[
            pltpu.VMEM((2, K), jnp.int32),
            pltpu.VMEM((K, 128), jnp.float32),
            pltpu.VMEM((64, 128), jnp.float32),
            pltpu.VMEM_SHARED((NACC, 128), jnp.float32),
        ],
    )
    def deg_kernel(ei_hbm, degp_hbm, idx_v, ones_v, zbuf, acc_sh):
        w = _wid()
        _zero_acc_slice(acc_sh, zbuf)
        _fill_value(ones_v, K, 1.0)
        plsc.subcore_barrier()

        def chunk(c, _):
            ebase = pl.multiple_of(w * (E // NW) + c * K, 8)
            pltpu.sync_copy(ei_hbm.at[pl.ds(ebase, K)], idx_v.at[0])
            pltpu.sync_copy(ei_hbm.at[pl.ds(E + ebase, K)], idx_v.at[1])
            pltpu.sync_copy(ones_v, acc_sh.at[idx_v.at[0]], add=True)
            pltpu.sync_copy(ones_v, acc_sh.at[idx_v.at[1]], add=True)
            return 0
        lax.fori_loop(0, NCHUNK, chunk, 0)

        plsc.subcore_barrier()
        _dump_acc_slice(acc_sh, zbuf, degp_hbm)

    return deg_kernel


def _make_nbr_kernel():
    @functools.partial(
        pl.kernel,
        out_type=jax.ShapeDtypeStruct((NC * NACC, 128), jnp.float32),
        mesh=_SC_MESH,
        scratch_types=[
            pltpu.VMEM((2, K), jnp.int32),
            pltpu.VMEM((K, 128), jnp.float32),
            pltpu.VMEM((K, 128), jnp.float32),
            pltpu.VMEM((64, 128), jnp.float32),
            pltpu.VMEM_SHARED((NACC, 128), jnp.float32),
            pltpu.SemaphoreType.DMA,
        ],
    )
    def nbr_kernel(ei_hbm, r1_hbm, accp_hbm, idx_v, rows0, rows1, zbuf,
                   acc_sh, sem):
        w = _wid()
        _zero_acc_slice(acc_sh, zbuf)
        plsc.subcore_barrier()

        def chunk(c, _):
            ebase = pl.multiple_of(w * (E // NW) + c * K, 8)
            pltpu.sync_copy(ei_hbm.at[pl.ds(ebase, K)], idx_v.at[0])
            pltpu.sync_copy(ei_hbm.at[pl.ds(E + ebase, K)], idx_v.at[1])
            cp0 = pltpu.async_copy(r1_hbm.at[idx_v.at[1]], rows0, sem)
            cp1 = pltpu.async_copy(r1_hbm.at[idx_v.at[0]], rows1, sem)
            cp0.wait()
            pltpu.sync_copy(rows0, acc_sh.at[idx_v.at[0]], add=True)
            cp1.wait()
            pltpu.sync_copy(rows1, acc_sh.at[idx_v.at[1]], add=True)
            return 0
        lax.fori_loop(0, NCHUNK, chunk, 0)

        plsc.subcore_barrier()
        _dump_acc_slice(acc_sh, zbuf, accp_hbm)

    return nbr_kernel


def _make_a3scat_kernel():
    @functools.partial(
        pl.kernel,
        out_type=jax.ShapeDtypeStruct((NC * NACC, 128), jnp.float32),
        mesh=_SC_MESH,
        scratch_types=[
            pltpu.VMEM((2, K), jnp.int32),
            pltpu.VMEM((K, 128), jnp.float32),
            pltpu.VMEM((64, 128), jnp.float32),
            pltpu.VMEM_SHARED((NACC, 128), jnp.float32),
        ],
    )
    def a3scat_kernel(ei_hbm, a3_hbm, zp_hbm, idx_v, rows, zbuf, acc_sh):
        w = _wid()
        _zero_acc_slice(acc_sh, zbuf)
        plsc.subcore_barrier()

        def chunk(c, _):
            ebase = pl.multiple_of(w * (E // NW) + c * K, 8)
            pltpu.sync_copy(ei_hbm.at[pl.ds(ebase, K)], idx_v.at[0])
            pltpu.sync_copy(ei_hbm.at[pl.ds(E + ebase, K)], idx_v.at[1])
            pltpu.sync_copy(a3_hbm.at[pl.ds(ebase, K)], rows)
            pltpu.sync_copy(rows, acc_sh.at[idx_v.at[0]], add=True)
            pltpu.sync_copy(rows, acc_sh.at[idx_v.at[1]], add=True)
            return 0
        lax.fori_loop(0, NCHUNK, chunk, 0)

        plsc.subcore_barrier()
        _dump_acc_slice(acc_sh, zbuf, zp_hbm)

    return a3scat_kernel


def _make_gath2_kernel():
    @functools.partial(
        pl.kernel,
        out_type=[jax.ShapeDtypeStruct((E, 128), jnp.float32),
                  jax.ShapeDtypeStruct((E, 128), jnp.float32)],
        mesh=_SC_MESH,
        scratch_types=[
            pltpu.VMEM((2, K), jnp.int32),
            pltpu.VMEM((K, 128), jnp.float32),
            pltpu.VMEM((K, 128), jnp.float32),
            pltpu.SemaphoreType.DMA,
        ],
    )
    def gath2_kernel(ei_hbm, t_hbm, outs_hbm, outd_hbm, idx_v, buf0, buf1,
                     sem):
        w = _wid()

        def chunk(c, _):
            ebase = pl.multiple_of(w * (E // NW) + c * K, 8)
            pltpu.sync_copy(ei_hbm.at[pl.ds(ebase, K)], idx_v.at[0])
            pltpu.sync_copy(ei_hbm.at[pl.ds(E + ebase, K)], idx_v.at[1])
            cp0 = pltpu.async_copy(t_hbm.at[idx_v.at[0]], buf0, sem)
            cp1 = pltpu.async_copy(t_hbm.at[idx_v.at[1]], buf1, sem)
            cp0.wait()
            pltpu.sync_copy(buf0, outs_hbm.at[pl.ds(ebase, K)])
            cp1.wait()
            pltpu.sync_copy(buf1, outd_hbm.at[pl.ds(ebase, K)])
            return 0
        lax.fori_loop(0, NCHUNK, chunk, 0)

    return gath2_kernel


KP = 40
NCHUNKP = (E // NW) // KP


def _make_gathp_kernel():
    @functools.partial(
        pl.kernel,
        out_type=[jax.ShapeDtypeStruct((E, 384), jnp.float32),
                  jax.ShapeDtypeStruct((E, 384), jnp.float32)],
        mesh=_SC_MESH,
        scratch_types=[
            pltpu.VMEM((2, KP), jnp.int32),
            pltpu.VMEM((KP, 384), jnp.float32),
            pltpu.VMEM((KP, 384), jnp.float32),
            pltpu.SemaphoreType.DMA,
        ],
    )
    def gathp_kernel(ei_hbm, t_hbm, outs_hbm, outd_hbm, idx_v, buf0, buf1,
                     sem):
        w = _wid()

        def chunk(c, _):
            ebase = pl.multiple_of(w * (E // NW) + c * KP, 8)
            pltpu.sync_copy(ei_hbm.at[pl.ds(ebase, KP)], idx_v.at[0])
            pltpu.sync_copy(ei_hbm.at[pl.ds(E + ebase, KP)], idx_v.at[1])
            cp0 = pltpu.async_copy(t_hbm.at[idx_v.at[0]], buf0, sem)
            cp1 = pltpu.async_copy(t_hbm.at[idx_v.at[1]], buf1, sem)
            cp0.wait()
            pltpu.sync_copy(buf0, outs_hbm.at[pl.ds(ebase, KP)])
            cp1.wait()
            pltpu.sync_copy(buf1, outd_hbm.at[pl.ds(ebase, KP)])
            return 0
        lax.fori_loop(0, NCHUNKP, chunk, 0)

    return gathp_kernel


def _make_gath6_kernel():
    @functools.partial(
        pl.kernel,
        out_type=[jax.ShapeDtypeStruct((E, 128), jnp.float32)
                  for _ in range(6)],
        mesh=_SC_MESH,
        scratch_types=[
            pltpu.VMEM((2, K), jnp.int32),
            pltpu.VMEM((K, 128), jnp.float32),
            pltpu.VMEM((K, 128), jnp.float32),
            pltpu.SemaphoreType.DMA,
        ],
    )
    def gath6_kernel(ei_hbm, ta_hbm, tb_hbm, tc_hbm, as_hbm, ad_hbm, bs_hbm,
                     bd_hbm, cs_hbm, cd_hbm, idx_v, buf0, buf1, sem):
        w = _wid()

        def chunk(c, _):
            ebase = pl.multiple_of(w * (E // NW) + c * K, 8)
            pltpu.sync_copy(ei_hbm.at[pl.ds(ebase, K)], idx_v.at[0])
            pltpu.sync_copy(ei_hbm.at[pl.ds(E + ebase, K)], idx_v.at[1])
            for t_hbm, os_hbm, od_hbm in ((ta_hbm, as_hbm, ad_hbm),
                                          (tb_hbm, bs_hbm, bd_hbm),
                                          (tc_hbm, cs_hbm, cd_hbm)):
                cp0 = pltpu.async_copy(t_hbm.at[idx_v.at[0]], buf0, sem)
                cp1 = pltpu.async_copy(t_hbm.at[idx_v.at[1]], buf1, sem)
                cp0.wait()
                pltpu.sync_copy(buf0, os_hbm.at[pl.ds(ebase, K)])
                cp1.wait()
                pltpu.sync_copy(buf1, od_hbm.at[pl.ds(ebase, K)])
            return 0
        lax.fori_loop(0, NCHUNK, chunk, 0)

    return gath6_kernel


def _node_mm_kernel(x_ref, won_ref, bon_ref, woff_ref, boff_ref, u_ref, v_ref):
    x = x_ref[...]
    u_ref[...] = x @ won_ref[...] + bon_ref[...]
    v_ref[...] = x @ woff_ref[...] + boff_ref[...]


def _dual_matmul(x, w_on, b_on, w_off, b_off, tile=1000):
    n = x.shape[0]
    d = x.shape[1]
    h = w_on.shape[1]
    grid = (n // tile,)
    return pl.pallas_call(
        _node_mm_kernel,
        grid=grid,
        in_specs=[
            pl.BlockSpec((tile, d), lambda i: (i, 0)),
            pl.BlockSpec((d, h), lambda i: (0, 0)),
            pl.BlockSpec((1, h), lambda i: (0, 0)),
            pl.BlockSpec((d, h), lambda i: (0, 0)),
            pl.BlockSpec((1, h), lambda i: (0, 0)),
        ],
        out_specs=[
            pl.BlockSpec((tile, h), lambda i: (i, 0)),
            pl.BlockSpec((tile, h), lambda i: (i, 0)),
        ],
        out_shape=[
            jax.ShapeDtypeStruct((n, h), jnp.float32),
            jax.ShapeDtypeStruct((n, h), jnp.float32),
        ],
    )(x, w_on, b_on[None, :], w_off, b_off[None, :])


def _p1q1r1_kernel(u_ref, v_ref, invdeg_ref, w2on_ref, w2off_ref,
                   q1h_ref, r1_ref):
    p1 = jax.nn.relu(u_ref[...] * invdeg_ref[...] + v_ref[...])
    q1h_ref[...] = (p1 @ w2on_ref[...]) * 0.5
    r1_ref[...] = p1 @ w2off_ref[...]


def _p1q1r1(u, v, invdeg, w2_on, w2_off, tile=1000):
    n, h = u.shape
    grid = (n // tile,)
    return pl.pallas_call(
        _p1q1r1_kernel,
        grid=grid,
        in_specs=[
            pl.BlockSpec((tile, h), lambda i: (i, 0)),
            pl.BlockSpec((tile, h), lambda i: (i, 0)),
            pl.BlockSpec((tile, h), lambda i: (i, 0)),
            pl.BlockSpec((h, h), lambda i: (0, 0)),
            pl.BlockSpec((h, h), lambda i: (0, 0)),
        ],
        out_specs=[
            pl.BlockSpec((tile, h), lambda i: (i, 0)),
            pl.BlockSpec((tile, h), lambda i: (i, 0)),
        ],
        out_shape=[
            jax.ShapeDtypeStruct((n, h), jnp.float32),
            jax.ShapeDtypeStruct((n, h), jnp.float32),
        ],
    )(u, v, invdeg, w2_on, w2_off)


def _stage2_kernel(gs_ref, gd_ref, b2on_ref, w3on_ref, b3on_ref, w3off_ref,
                   a3_ref, b3_ref):
    gs = gs_ref[...]
    gd = gd_ref[...]
    m = gs[:, 0:128] + gd[:, 0:128] + b2on_ref[...]
    u0 = (m + gs[:, 128:256]) * gs[:, 256:384]
    u1 = (m + gd[:, 128:256]) * gd[:, 256:384]
    g2 = (jax.nn.relu(u0) + jax.nn.relu(u1)) * 0.5
    a3_ref[...] = g2 @ w3off_ref[...]
    b3_ref[...] = g2 @ w3on_ref[...] + b3on_ref[...]


def _stage2(gs, gd, b2_on, w3_on, b3_on, w3_off, tile=1280):
    e = gs.shape[0]
    h = 128
    grid = (e // tile,)
    row = lambda i: (i, 0)
    fixed = lambda i: (0, 0)
    return pl.pallas_call(
        _stage2_kernel,
        grid=grid,
        in_specs=[
            pl.BlockSpec((tile, 384), row),
            pl.BlockSpec((tile, 384), row),
            pl.BlockSpec((1, h), fixed),
            pl.BlockSpec((h, h), fixed),
            pl.BlockSpec((1, h), fixed),
            pl.BlockSpec((h, h), fixed),
        ],
        out_specs=[
            pl.BlockSpec((tile, h), row),
            pl.BlockSpec((tile, h), row),
        ],
        out_shape=[
            jax.ShapeDtypeStruct((e, h), jnp.float32),
            jax.ShapeDtypeStruct((e, h), jnp.float32),
        ],
    )(gs, gd, b2_on[None, :], w3_on, b3_on[None, :], w3_off)


def _stage3_kernel(b3_ref, z3s_ref, z3d_ref, ws_ref, wd_ref, part_ref):
    i = pl.program_id(0)
    b3 = b3_ref[...]
    ws = ws_ref[...]
    wd = wd_ref[...]
    v0 = (b3 + z3s_ref[...]) * ws
    v1 = (b3 + z3d_ref[...]) * wd
    g3w = (jax.nn.relu(v0) + jax.nn.relu(v1)) * ((ws + wd) * 0.5)
    psum = jnp.sum(g3w.reshape(-1, 8, 128), axis=0)

    @pl.when(i == 0)
    def _():
        part_ref[...] = jnp.zeros_like(part_ref)
    part_ref[...] += psum


def _stage3(b3, z3s, z3d, gs, gd, tile=1280):
    e, h = b3.shape
    grid = (e // tile,)
    row = lambda i: (i, 0)
    wcol = lambda i: (i, 2)
    return pl.pallas_call(
        _stage3_kernel,
        grid=grid,
        in_specs=[
            pl.BlockSpec((tile, h), row),
            pl.BlockSpec((tile, h), row),
            pl.BlockSpec((tile, h), row),
            pl.BlockSpec((tile, h), wcol),
            pl.BlockSpec((tile, h), wcol),
        ],
        out_specs=pl.BlockSpec((8, h), lambda i: (0, 0)),
        out_shape=jax.ShapeDtypeStruct((8, h), jnp.float32),
    )(b3, z3s, z3d, gs, gd)


def kernel(x, edge_index, batch, w1_on, b1_on, w1_off, b1_off, w2_on, b2_on,
           w2_off, b2_off, w3_on, b3_on, w3_off, b3_off, wl1, bl1, wl2, bl2):
    n = x.shape[0]

    ei_flat = edge_index.reshape(-1)
    degp = _make_deg_kernel()(ei_flat)
    deg = degp[:N] + degp[NACC:NACC + N]
    invdeg = 1.0 / jnp.maximum(deg, 1.0)

    U, V = _dual_matmul(x, w1_on, b1_on, w1_off, b1_off)
    Q1h, R1 = _p1q1r1(U, V, invdeg, w2_on, w2_off)

    accp = _make_nbr_kernel()(ei_flat, R1)
    acc = accp[:N] + accp[NACC:NACC + N]
    z2 = deg * b2_off + (deg * R1 + acc) * 0.5

    packed = jnp.concatenate([Q1h, z2, invdeg], axis=1)
    Gs, Gd = _make_gathp_kernel()(ei_flat, packed)
    A3, B3 = _stage2(Gs, Gd, b2_on, w3_on, b3_on, w3_off)

    z3p = _make_a3scat_kernel()(ei_flat, A3)
    z3 = z3p[:N] + z3p[NACC:NACC + N] + deg * b3_off
    z3s, z3d = _make_gath2_kernel()(ei_flat, z3)

    parts = _stage3(B3, z3s, z3d, Gs, Gd)
    total = jnp.sum(parts, axis=0)
    g = (total / n)[None, :]

    h1 = jax.nn.relu(g @ wl1 + bl1)
    logits = h1 @ wl2 + bl2
    return jax.nn.log_softmax(logits, axis=-1)


# speedup vs baseline: 7.4270x; 1.0247x over previous
import functools
import jax
import jax.numpy as jnp
from jax import lax
from jax.experimental import pallas as pl
from jax.experimental.pallas import tpu as pltpu
from jax.experimental.pallas import tpu_sc as plsc

N = 10000
E = 320000
H = 128
NC = 2
NS = 16
NW = NC * NS
K = 80
NCHUNK = (E // NW) // K
NACC = 10240
SLICE = NACC // NS

_SC_MESH = plsc.VectorSubcoreMesh(
    core_axis_name="c", subcore_axis_name="s", num_cores=NC, num_subcores=NS)


def _wid():
    return lax.axis_index("s") * NC + lax.axis_index("c")


def _fill_value(ref, rows, value):
    vals = jnp.full((16,), value, jnp.float32)
    def body(i, _):
        r = i // 8
        h = i % 8
        ref[r, pl.ds(h * 16, 16)] = vals
        return 0
    lax.fori_loop(0, rows * 8, body, 0)


def _zero_acc_slice(acc_sh, zbuf):
    sid = lax.axis_index("s")
    zrows = zbuf.shape[0]
    _fill_value(zbuf, zrows, 0.0)
    base = pl.multiple_of(sid * SLICE, 8)
    for j in range(SLICE // zrows):
        pltpu.sync_copy(zbuf, acc_sh.at[pl.ds(base + j * zrows, zrows)])


def _dump_acc_slice(acc_sh, dbuf, out_hbm):
    cid = lax.axis_index("c")
    sid = lax.axis_index("s")
    drows = dbuf.shape[0]
    base = pl.multiple_of(sid * SLICE, 8)
    obase = pl.multiple_of(cid * NACC + base, 8)
    for j in range(SLICE // drows):
        pltpu.sync_copy(acc_sh.at[pl.ds(base + j * drows, drows)], dbuf)
        pltpu.sync_copy(dbuf, out_hbm.at[pl.ds(obase + j * drows, drows)])


def _make_deg_kernel():
    @functools.partial(
        pl.kernel,
        out_type=jax.ShapeDtypeStruct((NC * NACC, 128), jnp.float32),
        mesh=_SC_MESH,
        scratch_types=[
            pltpu.VMEM((2, K), jnp.int32),
            pltpu.VMEM((K, 128), jnp.float32),
            pltpu.VMEM((64, 128), jnp.float32),
            pltpu.VMEM_SHARED((NACC, 128), jnp.float32),
        ],
    )
    def deg_kernel(ei_hbm, degp_hbm, idx_v, ones_v, zbuf, acc_sh):
        w = _wid()
        _zero_acc_slice(acc_sh, zbuf)
        _fill_value(ones_v, K, 1.0)
        plsc.subcore_barrier()

        def chunk(c, _):
            ebase = pl.multiple_of(w * (E // NW) + c * K, 8)
            pltpu.sync_copy(ei_hbm.at[pl.ds(ebase, K)], idx_v.at[0])
            pltpu.sync_copy(ei_hbm.at[pl.ds(E + ebase, K)], idx_v.at[1])
            pltpu.sync_copy(ones_v, acc_sh.at[idx_v.at[0]], add=True)
            pltpu.sync_copy(ones_v, acc_sh.at[idx_v.at[1]], add=True)
            return 0
        lax.fori_loop(0, NCHUNK, chunk, 0)

        plsc.subcore_barrier()
        _dump_acc_slice(acc_sh, zbuf, degp_hbm)

    return deg_kernel


def _make_nbr_kernel():
    @functools.partial(
        pl.kernel,
        out_type=jax.ShapeDtypeStruct((NC * NACC, 128), jnp.float32),
        mesh=_SC_MESH,
        scratch_types=[
            pltpu.VMEM((2, K), jnp.int32),
            pltpu.VMEM((K, 128), jnp.float32),
            pltpu.VMEM((K, 128), jnp.float32),
            pltpu.VMEM((64, 128), jnp.float32),
            pltpu.VMEM_SHARED((NACC, 128), jnp.float32),
            pltpu.SemaphoreType.DMA,
        ],
    )
    def nbr_kernel(ei_hbm, r1_hbm, accp_hbm, idx_v, rows0, rows1, zbuf,
                   acc_sh, sem):
        w = _wid()
        _zero_acc_slice(acc_sh, zbuf)
        plsc.subcore_barrier()

        def chunk(c, _):
            ebase = pl.multiple_of(w * (E // NW) + c * K, 8)
            pltpu.sync_copy(ei_hbm.at[pl.ds(ebase, K)], idx_v.at[0])
            pltpu.sync_copy(ei_hbm.at[pl.ds(E + ebase, K)], idx_v.at[1])
            cp0 = pltpu.async_copy(r1_hbm.at[idx_v.at[1]], rows0, sem)
            cp1 = pltpu.async_copy(r1_hbm.at[idx_v.at[0]], rows1, sem)
            cp0.wait()
            pltpu.sync_copy(rows0, acc_sh.at[idx_v.at[0]], add=True)
            cp1.wait()
            pltpu.sync_copy(rows1, acc_sh.at[idx_v.at[1]], add=True)
            return 0
        lax.fori_loop(0, NCHUNK, chunk, 0)

        plsc.subcore_barrier()
        _dump_acc_slice(acc_sh, zbuf, accp_hbm)

    return nbr_kernel


def _make_a3scat_kernel():
    @functools.partial(
        pl.kernel,
        out_type=jax.ShapeDtypeStruct((NC * NACC, 128), jnp.float32),
        mesh=_SC_MESH,
        scratch_types=[
            pltpu.VMEM((2, K), jnp.int32),
            pltpu.VMEM((K, 128), jnp.float32),
            pltpu.VMEM((64, 128), jnp.float32),
            pltpu.VMEM_SHARED((NACC, 128), jnp.float32),
        ],
    )
    def a3scat_kernel(ei_hbm, a3_hbm, zp_hbm, idx_v, rows, zbuf, acc_sh):
        w = _wid()
        _zero_acc_slice(acc_sh, zbuf)
        plsc.subcore_barrier()

        def chunk(c, _):
            ebase = pl.multiple_of(w * (E // NW) + c * K, 8)
            pltpu.sync_copy(ei_hbm.at[pl.ds(ebase, K)], idx_v.at[0])
            pltpu.sync_copy(ei_hbm.at[pl.ds(E + ebase, K)], idx_v.at[1])
            pltpu.sync_copy(a3_hbm.at[pl.ds(ebase, K)], rows)
            pltpu.sync_copy(rows, acc_sh.at[idx_v.at[0]], add=True)
            pltpu.sync_copy(rows, acc_sh.at[idx_v.at[1]], add=True)
            return 0
        lax.fori_loop(0, NCHUNK, chunk, 0)

        plsc.subcore_barrier()
        _dump_acc_slice(acc_sh, zbuf, zp_hbm)

    return a3scat_kernel


def _make_gath2_kernel():
    @functools.partial(
        pl.kernel,
        out_type=[jax.ShapeDtypeStruct((E, 128), jnp.float32),
                  jax.ShapeDtypeStruct((E, 128), jnp.float32)],
        mesh=_SC_MESH,
        scratch_types=[
            pltpu.VMEM((2, K), jnp.int32),
            pltpu.VMEM((K, 128), jnp.float32),
            pltpu.VMEM((K, 128), jnp.float32),
            pltpu.SemaphoreType.DMA,
        ],
    )
    def gath2_kernel(ei_hbm, t_hbm, outs_hbm, outd_hbm, idx_v, buf0, buf1,
                     sem):
        w = _wid()

        def chunk(c, _):
            ebase = pl.multiple_of(w * (E // NW) + c * K, 8)
            pltpu.sync_copy(ei_hbm.at[pl.ds(ebase, K)], idx_v.at[0])
            pltpu.sync_copy(ei_hbm.at[pl.ds(E + ebase, K)], idx_v.at[1])
            cp0 = pltpu.async_copy(t_hbm.at[idx_v.at[0]], buf0, sem)
            cp1 = pltpu.async_copy(t_hbm.at[idx_v.at[1]], buf1, sem)
            cp0.wait()
            pltpu.sync_copy(buf0, outs_hbm.at[pl.ds(ebase, K)])
            cp1.wait()
            pltpu.sync_copy(buf1, outd_hbm.at[pl.ds(ebase, K)])
            return 0
        lax.fori_loop(0, NCHUNK, chunk, 0)

    return gath2_kernel


KP = 40
NCHUNKP = (E // NW) // KP


def _make_gathp_kernel():
    @functools.partial(
        pl.kernel,
        out_type=[jax.ShapeDtypeStruct((E, 384), jnp.float32),
                  jax.ShapeDtypeStruct((E, 384), jnp.float32)],
        mesh=_SC_MESH,
        scratch_types=[
            pltpu.VMEM((2, KP), jnp.int32),
            pltpu.VMEM((KP, 384), jnp.float32),
            pltpu.VMEM((KP, 384), jnp.float32),
            pltpu.SemaphoreType.DMA,
        ],
    )
    def gathp_kernel(ei_hbm, t_hbm, outs_hbm, outd_hbm, idx_v, buf0, buf1,
                     sem):
        w = _wid()

        def chunk(c, _):
            ebase = pl.multiple_of(w * (E // NW) + c * KP, 8)
            pltpu.sync_copy(ei_hbm.at[pl.ds(ebase, KP)], idx_v.at[0])
            pltpu.sync_copy(ei_hbm.at[pl.ds(E + ebase, KP)], idx_v.at[1])
            cp0 = pltpu.async_copy(t_hbm.at[idx_v.at[0]], buf0, sem)
            cp1 = pltpu.async_copy(t_hbm.at[idx_v.at[1]], buf1, sem)
            cp0.wait()
            pltpu.sync_copy(buf0, outs_hbm.at[pl.ds(ebase, KP)])
            cp1.wait()
            pltpu.sync_copy(buf1, outd_hbm.at[pl.ds(ebase, KP)])
            return 0
        lax.fori_loop(0, NCHUNKP, chunk, 0)

    return gathp_kernel


def _make_gath6_kernel():
    @functools.partial(
        pl.kernel,
        out_type=[jax.ShapeDtypeStruct((E, 128), jnp.float32)
                  for _ in range(6)],
        mesh=_SC_MESH,
        scratch_types=[
            pltpu.VMEM((2, K), jnp.int32),
            pltpu.VMEM((K, 128), jnp.float32),
            pltpu.VMEM((K, 128), jnp.float32),
            pltpu.SemaphoreType.DMA,
        ],
    )
    def gath6_kernel(ei_hbm, ta_hbm, tb_hbm, tc_hbm, as_hbm, ad_hbm, bs_hbm,
                     bd_hbm, cs_hbm, cd_hbm, idx_v, buf0, buf1, sem):
        w = _wid()

        def chunk(c, _):
            ebase = pl.multiple_of(w * (E // NW) + c * K, 8)
            pltpu.sync_copy(ei_hbm.at[pl.ds(ebase, K)], idx_v.at[0])
            pltpu.sync_copy(ei_hbm.at[pl.ds(E + ebase, K)], idx_v.at[1])
            for t_hbm, os_hbm, od_hbm in ((ta_hbm, as_hbm, ad_hbm),
                                          (tb_hbm, bs_hbm, bd_hbm),
                                          (tc_hbm, cs_hbm, cd_hbm)):
                cp0 = pltpu.async_copy(t_hbm.at[idx_v.at[0]], buf0, sem)
                cp1 = pltpu.async_copy(t_hbm.at[idx_v.at[1]], buf1, sem)
                cp0.wait()
                pltpu.sync_copy(buf0, os_hbm.at[pl.ds(ebase, K)])
                cp1.wait()
                pltpu.sync_copy(buf1, od_hbm.at[pl.ds(ebase, K)])
            return 0
        lax.fori_loop(0, NCHUNK, chunk, 0)

    return gath6_kernel


def _node_mm_kernel(x_ref, won_ref, bon_ref, woff_ref, boff_ref, u_ref, v_ref):
    x = x_ref[...]
    u_ref[...] = x @ won_ref[...] + bon_ref[...]
    v_ref[...] = x @ woff_ref[...] + boff_ref[...]


def _dual_matmul(x, w_on, b_on, w_off, b_off, tile=1000):
    n = x.shape[0]
    d = x.shape[1]
    h = w_on.shape[1]
    grid = (n // tile,)
    return pl.pallas_call(
        _node_mm_kernel,
        grid=grid,
        in_specs=[
            pl.BlockSpec((tile, d), lambda i: (i, 0)),
            pl.BlockSpec((d, h), lambda i: (0, 0)),
            pl.BlockSpec((1, h), lambda i: (0, 0)),
            pl.BlockSpec((d, h), lambda i: (0, 0)),
            pl.BlockSpec((1, h), lambda i: (0, 0)),
        ],
        out_specs=[
            pl.BlockSpec((tile, h), lambda i: (i, 0)),
            pl.BlockSpec((tile, h), lambda i: (i, 0)),
        ],
        out_shape=[
            jax.ShapeDtypeStruct((n, h), jnp.float32),
            jax.ShapeDtypeStruct((n, h), jnp.float32),
        ],
    )(x, w_on, b_on[None, :], w_off, b_off[None, :])


def _p1q1r1_kernel(u_ref, v_ref, invdeg_ref, w2on_ref, w2off_ref,
                   q1h_ref, r1_ref):
    p1 = jax.nn.relu(u_ref[...] * invdeg_ref[...] + v_ref[...])
    q1h_ref[...] = (p1 @ w2on_ref[...]) * 0.5
    r1_ref[...] = p1 @ w2off_ref[...]


def _p1q1r1(u, v, invdeg, w2_on, w2_off, tile=1000):
    n, h = u.shape
    grid = (n // tile,)
    return pl.pallas_call(
        _p1q1r1_kernel,
        grid=grid,
        in_specs=[
            pl.BlockSpec((tile, h), lambda i: (i, 0)),
            pl.BlockSpec((tile, h), lambda i: (i, 0)),
            pl.BlockSpec((tile, h), lambda i: (i, 0)),
            pl.BlockSpec((h, h), lambda i: (0, 0)),
            pl.BlockSpec((h, h), lambda i: (0, 0)),
        ],
        out_specs=[
            pl.BlockSpec((tile, h), lambda i: (i, 0)),
            pl.BlockSpec((tile, h), lambda i: (i, 0)),
        ],
        out_shape=[
            jax.ShapeDtypeStruct((n, h), jnp.float32),
            jax.ShapeDtypeStruct((n, h), jnp.float32),
        ],
    )(u, v, invdeg, w2_on, w2_off)


def _stage2_kernel(gs_ref, gd_ref, b2on_ref, w3on_ref, b3on_ref, w3off_ref,
                   a3_ref, b3_ref):
    gs = gs_ref[...]
    gd = gd_ref[...]
    m = gs[:, 0:128] + gd[:, 0:128] + b2on_ref[...]
    u0 = (m + gs[:, 128:256]) * gs[:, 256:384]
    u1 = (m + gd[:, 128:256]) * gd[:, 256:384]
    g2 = (jax.nn.relu(u0) + jax.nn.relu(u1)) * 0.5
    a3_ref[...] = g2 @ w3off_ref[...]
    b3_ref[...] = g2 @ w3on_ref[...] + b3on_ref[...]


def _stage2(gs, gd, b2_on, w3_on, b3_on, w3_off, tile=2560):
    e = gs.shape[0]
    h = 128
    grid = (e // tile,)
    row = lambda i: (i, 0)
    fixed = lambda i: (0, 0)
    return pl.pallas_call(
        _stage2_kernel,
        grid=grid,
        in_specs=[
            pl.BlockSpec((tile, 384), row),
            pl.BlockSpec((tile, 384), row),
            pl.BlockSpec((1, h), fixed),
            pl.BlockSpec((h, h), fixed),
            pl.BlockSpec((1, h), fixed),
            pl.BlockSpec((h, h), fixed),
        ],
        out_specs=[
            pl.BlockSpec((tile, h), row),
            pl.BlockSpec((tile, h), row),
        ],
        out_shape=[
            jax.ShapeDtypeStruct((e, h), jnp.float32),
            jax.ShapeDtypeStruct((e, h), jnp.float32),
        ],
    )(gs, gd, b2_on[None, :], w3_on, b3_on[None, :], w3_off)


def _stage3_kernel(b3_ref, z3s_ref, z3d_ref, ws_ref, wd_ref, part_ref):
    i = pl.program_id(0)
    b3 = b3_ref[...]
    ws = ws_ref[...]
    wd = wd_ref[...]
    v0 = (b3 + z3s_ref[...]) * ws
    v1 = (b3 + z3d_ref[...]) * wd
    g3w = (jax.nn.relu(v0) + jax.nn.relu(v1)) * ((ws + wd) * 0.5)
    psum = jnp.sum(g3w.reshape(-1, 8, 128), axis=0)

    @pl.when(i == 0)
    def _():
        part_ref[...] = jnp.zeros_like(part_ref)
    part_ref[...] += psum


def _stage3(b3, z3s, z3d, gs, gd, tile=2560):
    e, h = b3.shape
    grid = (e // tile,)
    row = lambda i: (i, 0)
    wcol = lambda i: (i, 2)
    return pl.pallas_call(
        _stage3_kernel,
        grid=grid,
        in_specs=[
            pl.BlockSpec((tile, h), row),
            pl.BlockSpec((tile, h), row),
            pl.BlockSpec((tile, h), row),
            pl.BlockSpec((tile, h), wcol),
            pl.BlockSpec((tile, h), wcol),
        ],
        out_specs=pl.BlockSpec((8, h), lambda i: (0, 0)),
        out_shape=jax.ShapeDtypeStruct((8, h), jnp.float32),
    )(b3, z3s, z3d, gs, gd)


def kernel(x, edge_index, batch, w1_on, b1_on, w1_off, b1_off, w2_on, b2_on,
           w2_off, b2_off, w3_on, b3_on, w3_off, b3_off, wl1, bl1, wl2, bl2):
    n = x.shape[0]

    ei_flat = edge_index.reshape(-1)
    degp = _make_deg_kernel()(ei_flat)
    deg = degp[:N] + degp[NACC:NACC + N]
    invdeg = 1.0 / jnp.maximum(deg, 1.0)

    U, V = _dual_matmul(x, w1_on, b1_on, w1_off, b1_off)
    Q1h, R1 = _p1q1r1(U, V, invdeg, w2_on, w2_off)

    accp = _make_nbr_kernel()(ei_flat, R1)
    acc = accp[:N] + accp[NACC:NACC + N]
    z2 = deg * b2_off + (deg * R1 + acc) * 0.5

    packed = jnp.concatenate([Q1h, z2, invdeg], axis=1)
    Gs, Gd = _make_gathp_kernel()(ei_flat, packed)
    A3, B3 = _stage2(Gs, Gd, b2_on, w3_on, b3_on, w3_off)

    z3p = _make_a3scat_kernel()(ei_flat, A3)
    z3 = z3p[:N] + z3p[NACC:NACC + N] + deg * b3_off
    z3s, z3d = _make_gath2_kernel()(ei_flat, z3)

    parts = _stage3(B3, z3s, z3d, Gs, Gd)
    total = jnp.sum(parts, axis=0)
    g = (total / n)[None, :]

    h1 = jax.nn.relu(g @ wl1 + bl1)
    logits = h1 @ wl2 + bl2
    return jax.nn.log_softmax(logits, axis=-1)


# speedup vs baseline: 7.9899x; 1.0758x over previous
import functools
import jax
import jax.numpy as jnp
from jax import lax
from jax.experimental import pallas as pl
from jax.experimental.pallas import tpu as pltpu
from jax.experimental.pallas import tpu_sc as plsc

N = 10000
E = 320000
H = 128
NC = 2
NS = 16
NW = NC * NS
K = 80
NCHUNK = (E // NW) // K
NACC = 10240
SLICE = NACC // NS

_SC_MESH = plsc.VectorSubcoreMesh(
    core_axis_name="c", subcore_axis_name="s", num_cores=NC, num_subcores=NS)


def _wid():
    return lax.axis_index("s") * NC + lax.axis_index("c")


def _fill_value(ref, rows, value):
    vals = jnp.full((16,), value, jnp.float32)
    def body(i, _):
        r = i // 8
        h = i % 8
        ref[r, pl.ds(h * 16, 16)] = vals
        return 0
    lax.fori_loop(0, rows * 8, body, 0)


def _zero_acc_slice(acc_sh, zbuf):
    sid = lax.axis_index("s")
    zrows = zbuf.shape[0]
    _fill_value(zbuf, zrows, 0.0)
    base = pl.multiple_of(sid * SLICE, 8)
    for j in range(SLICE // zrows):
        pltpu.sync_copy(zbuf, acc_sh.at[pl.ds(base + j * zrows, zrows)])


def _dump_acc_slice(acc_sh, dbuf, out_hbm):
    cid = lax.axis_index("c")
    sid = lax.axis_index("s")
    drows = dbuf.shape[0]
    base = pl.multiple_of(sid * SLICE, 8)
    obase = pl.multiple_of(cid * NACC + base, 8)
    for j in range(SLICE // drows):
        pltpu.sync_copy(acc_sh.at[pl.ds(base + j * drows, drows)], dbuf)
        pltpu.sync_copy(dbuf, out_hbm.at[pl.ds(obase + j * drows, drows)])


def _make_deg_kernel():
    @functools.partial(
        pl.kernel,
        out_type=jax.ShapeDtypeStruct((NC * NACC, 128), jnp.float32),
        mesh=_SC_MESH,
        scratch_types=[
            pltpu.VMEM((2, K), jnp.int32),
            pltpu.VMEM((K, 128), jnp.float32),
            pltpu.VMEM((64, 128), jnp.float32),
            pltpu.VMEM_SHARED((NACC, 128), jnp.float32),
        ],
    )
    def deg_kernel(ei_hbm, degp_hbm, idx_v, ones_v, zbuf, acc_sh):
        w = _wid()
        _zero_acc_slice(acc_sh, zbuf)
        _fill_value(ones_v, K, 1.0)
        plsc.subcore_barrier()

        def chunk(c, _):
            ebase = pl.multiple_of(w * (E // NW) + c * K, 8)
            pltpu.sync_copy(ei_hbm.at[pl.ds(ebase, K)], idx_v.at[0])
            pltpu.sync_copy(ei_hbm.at[pl.ds(E + ebase, K)], idx_v.at[1])
            pltpu.sync_copy(ones_v, acc_sh.at[idx_v.at[0]], add=True)
            pltpu.sync_copy(ones_v, acc_sh.at[idx_v.at[1]], add=True)
            return 0
        lax.fori_loop(0, NCHUNK, chunk, 0)

        plsc.subcore_barrier()
        _dump_acc_slice(acc_sh, zbuf, degp_hbm)

    return deg_kernel


def _make_nbr_kernel():
    @functools.partial(
        pl.kernel,
        out_type=jax.ShapeDtypeStruct((NC * NACC, 128), jnp.float32),
        mesh=_SC_MESH,
        scratch_types=[
            pltpu.VMEM((2, K), jnp.int32),
            pltpu.VMEM((K, 128), jnp.float32),
            pltpu.VMEM((K, 128), jnp.float32),
            pltpu.VMEM((64, 128), jnp.float32),
            pltpu.VMEM_SHARED((NACC, 128), jnp.float32),
            pltpu.SemaphoreType.DMA,
        ],
    )
    def nbr_kernel(ei_hbm, r1_hbm, accp_hbm, idx_v, rows0, rows1, zbuf,
                   acc_sh, sem):
        w = _wid()
        _zero_acc_slice(acc_sh, zbuf)
        plsc.subcore_barrier()

        def chunk(c, _):
            ebase = pl.multiple_of(w * (E // NW) + c * K, 8)
            pltpu.sync_copy(ei_hbm.at[pl.ds(ebase, K)], idx_v.at[0])
            pltpu.sync_copy(ei_hbm.at[pl.ds(E + ebase, K)], idx_v.at[1])
            cp0 = pltpu.async_copy(r1_hbm.at[idx_v.at[1]], rows0, sem)
            cp1 = pltpu.async_copy(r1_hbm.at[idx_v.at[0]], rows1, sem)
            cp0.wait()
            pltpu.sync_copy(rows0, acc_sh.at[idx_v.at[0]], add=True)
            cp1.wait()
            pltpu.sync_copy(rows1, acc_sh.at[idx_v.at[1]], add=True)
            return 0
        lax.fori_loop(0, NCHUNK, chunk, 0)

        plsc.subcore_barrier()
        _dump_acc_slice(acc_sh, zbuf, accp_hbm)

    return nbr_kernel


def _make_a3scat_kernel():
    @functools.partial(
        pl.kernel,
        out_type=jax.ShapeDtypeStruct((NC * NACC, 128), jnp.float32),
        mesh=_SC_MESH,
        scratch_types=[
            pltpu.VMEM((2, K), jnp.int32),
            pltpu.VMEM((K, 128), jnp.float32),
            pltpu.VMEM((64, 128), jnp.float32),
            pltpu.VMEM_SHARED((NACC, 128), jnp.float32),
        ],
    )
    def a3scat_kernel(ei_hbm, a3_hbm, zp_hbm, idx_v, rows, zbuf, acc_sh):
        w = _wid()
        _zero_acc_slice(acc_sh, zbuf)
        plsc.subcore_barrier()

        def chunk(c, _):
            ebase = pl.multiple_of(w * (E // NW) + c * K, 8)
            pltpu.sync_copy(ei_hbm.at[pl.ds(ebase, K)], idx_v.at[0])
            pltpu.sync_copy(ei_hbm.at[pl.ds(E + ebase, K)], idx_v.at[1])
            pltpu.sync_copy(a3_hbm.at[pl.ds(ebase, K)], rows)
            pltpu.sync_copy(rows, acc_sh.at[idx_v.at[0]], add=True)
            pltpu.sync_copy(rows, acc_sh.at[idx_v.at[1]], add=True)
            return 0
        lax.fori_loop(0, NCHUNK, chunk, 0)

        plsc.subcore_barrier()
        _dump_acc_slice(acc_sh, zbuf, zp_hbm)

    return a3scat_kernel


def _make_gath2_kernel():
    @functools.partial(
        pl.kernel,
        out_type=[jax.ShapeDtypeStruct((E, 128), jnp.float32),
                  jax.ShapeDtypeStruct((E, 128), jnp.float32)],
        mesh=_SC_MESH,
        scratch_types=[
            pltpu.VMEM((2, K), jnp.int32),
            pltpu.VMEM((K, 128), jnp.float32),
            pltpu.VMEM((K, 128), jnp.float32),
            pltpu.SemaphoreType.DMA,
        ],
    )
    def gath2_kernel(ei_hbm, t_hbm, outs_hbm, outd_hbm, idx_v, buf0, buf1,
                     sem):
        w = _wid()

        def chunk(c, _):
            ebase = pl.multiple_of(w * (E // NW) + c * K, 8)
            pltpu.sync_copy(ei_hbm.at[pl.ds(ebase, K)], idx_v.at[0])
            pltpu.sync_copy(ei_hbm.at[pl.ds(E + ebase, K)], idx_v.at[1])
            cp0 = pltpu.async_copy(t_hbm.at[idx_v.at[0]], buf0, sem)
            cp1 = pltpu.async_copy(t_hbm.at[idx_v.at[1]], buf1, sem)
            cp0.wait()
            pltpu.sync_copy(buf0, outs_hbm.at[pl.ds(ebase, K)])
            cp1.wait()
            pltpu.sync_copy(buf1, outd_hbm.at[pl.ds(ebase, K)])
            return 0
        lax.fori_loop(0, NCHUNK, chunk, 0)

    return gath2_kernel


KP = 80
NCHUNKP = (E // NW) // KP


def _make_gathp_kernel():
    @functools.partial(
        pl.kernel,
        out_type=[jax.ShapeDtypeStruct((E, 384), jnp.float32),
                  jax.ShapeDtypeStruct((E, 384), jnp.float32)],
        mesh=_SC_MESH,
        scratch_types=[
            pltpu.VMEM((2, KP), jnp.int32),
            pltpu.VMEM((KP, 384), jnp.float32),
            pltpu.VMEM((KP, 384), jnp.float32),
            pltpu.SemaphoreType.DMA,
        ],
    )
    def gathp_kernel(ei_hbm, t_hbm, outs_hbm, outd_hbm, idx_v, buf0, buf1,
                     sem):
        w = _wid()

        def chunk(c, _):
            ebase = pl.multiple_of(w * (E // NW) + c * KP, 8)
            pltpu.sync_copy(ei_hbm.at[pl.ds(ebase, KP)], idx_v.at[0])
            pltpu.sync_copy(ei_hbm.at[pl.ds(E + ebase, KP)], idx_v.at[1])
            cp0 = pltpu.async_copy(t_hbm.at[idx_v.at[0]], buf0, sem)
            cp1 = pltpu.async_copy(t_hbm.at[idx_v.at[1]], buf1, sem)
            cp0.wait()
            pltpu.sync_copy(buf0, outs_hbm.at[pl.ds(ebase, KP)])
            cp1.wait()
            pltpu.sync_copy(buf1, outd_hbm.at[pl.ds(ebase, KP)])
            return 0
        lax.fori_loop(0, NCHUNKP, chunk, 0)

    return gathp_kernel


def _make_gath6_kernel():
    @functools.partial(
        pl.kernel,
        out_type=[jax.ShapeDtypeStruct((E, 128), jnp.float32)
                  for _ in range(6)],
        mesh=_SC_MESH,
        scratch_types=[
            pltpu.VMEM((2, K), jnp.int32),
            pltpu.VMEM((K, 128), jnp.float32),
            pltpu.VMEM((K, 128), jnp.float32),
            pltpu.SemaphoreType.DMA,
        ],
    )
    def gath6_kernel(ei_hbm, ta_hbm, tb_hbm, tc_hbm, as_hbm, ad_hbm, bs_hbm,
                     bd_hbm, cs_hbm, cd_hbm, idx_v, buf0, buf1, sem):
        w = _wid()

        def chunk(c, _):
            ebase = pl.multiple_of(w * (E // NW) + c * K, 8)
            pltpu.sync_copy(ei_hbm.at[pl.ds(ebase, K)], idx_v.at[0])
            pltpu.sync_copy(ei_hbm.at[pl.ds(E + ebase, K)], idx_v.at[1])
            for t_hbm, os_hbm, od_hbm in ((ta_hbm, as_hbm, ad_hbm),
                                          (tb_hbm, bs_hbm, bd_hbm),
                                          (tc_hbm, cs_hbm, cd_hbm)):
                cp0 = pltpu.async_copy(t_hbm.at[idx_v.at[0]], buf0, sem)
                cp1 = pltpu.async_copy(t_hbm.at[idx_v.at[1]], buf1, sem)
                cp0.wait()
                pltpu.sync_copy(buf0, os_hbm.at[pl.ds(ebase, K)])
                cp1.wait()
                pltpu.sync_copy(buf1, od_hbm.at[pl.ds(ebase, K)])
            return 0
        lax.fori_loop(0, NCHUNK, chunk, 0)

    return gath6_kernel


def _node_mm_kernel(x_ref, won_ref, bon_ref, woff_ref, boff_ref, u_ref, v_ref):
    x = x_ref[...]
    u_ref[...] = x @ won_ref[...] + bon_ref[...]
    v_ref[...] = x @ woff_ref[...] + boff_ref[...]


def _dual_matmul(x, w_on, b_on, w_off, b_off, tile=1000):
    n = x.shape[0]
    d = x.shape[1]
    h = w_on.shape[1]
    grid = (n // tile,)
    return pl.pallas_call(
        _node_mm_kernel,
        grid=grid,
        in_specs=[
            pl.BlockSpec((tile, d), lambda i: (i, 0)),
            pl.BlockSpec((d, h), lambda i: (0, 0)),
            pl.BlockSpec((1, h), lambda i: (0, 0)),
            pl.BlockSpec((d, h), lambda i: (0, 0)),
            pl.BlockSpec((1, h), lambda i: (0, 0)),
        ],
        out_specs=[
            pl.BlockSpec((tile, h), lambda i: (i, 0)),
            pl.BlockSpec((tile, h), lambda i: (i, 0)),
        ],
        out_shape=[
            jax.ShapeDtypeStruct((n, h), jnp.float32),
            jax.ShapeDtypeStruct((n, h), jnp.float32),
        ],
    )(x, w_on, b_on[None, :], w_off, b_off[None, :])


def _p1q1r1_kernel(u_ref, v_ref, invdeg_ref, w2on_ref, w2off_ref,
                   q1h_ref, r1_ref):
    p1 = jax.nn.relu(u_ref[...] * invdeg_ref[...] + v_ref[...])
    q1h_ref[...] = (p1 @ w2on_ref[...]) * 0.5
    r1_ref[...] = p1 @ w2off_ref[...]


def _p1q1r1(u, v, invdeg, w2_on, w2_off, tile=1000):
    n, h = u.shape
    grid = (n // tile,)
    return pl.pallas_call(
        _p1q1r1_kernel,
        grid=grid,
        in_specs=[
            pl.BlockSpec((tile, h), lambda i: (i, 0)),
            pl.BlockSpec((tile, h), lambda i: (i, 0)),
            pl.BlockSpec((tile, h), lambda i: (i, 0)),
            pl.BlockSpec((h, h), lambda i: (0, 0)),
            pl.BlockSpec((h, h), lambda i: (0, 0)),
        ],
        out_specs=[
            pl.BlockSpec((tile, h), lambda i: (i, 0)),
            pl.BlockSpec((tile, h), lambda i: (i, 0)),
        ],
        out_shape=[
            jax.ShapeDtypeStruct((n, h), jnp.float32),
            jax.ShapeDtypeStruct((n, h), jnp.float32),
        ],
    )(u, v, invdeg, w2_on, w2_off)


def _stage2_kernel(gs_ref, gd_ref, b2on_ref, w3on_ref, b3on_ref, w3off_ref,
                   a3_ref, b3_ref):
    gs = gs_ref[...]
    gd = gd_ref[...]
    m = gs[:, 0:128] + gd[:, 0:128] + b2on_ref[...]
    u0 = (m + gs[:, 128:256]) * gs[:, 256:384]
    u1 = (m + gd[:, 128:256]) * gd[:, 256:384]
    g2 = (jax.nn.relu(u0) + jax.nn.relu(u1)) * 0.5
    a3_ref[...] = g2 @ w3off_ref[...]
    b3_ref[...] = g2 @ w3on_ref[...] + b3on_ref[...]


def _stage2(gs, gd, b2_on, w3_on, b3_on, w3_off, tile=2560):
    e = gs.shape[0]
    h = 128
    grid = (e // tile,)
    row = lambda i: (i, 0)
    fixed = lambda i: (0, 0)
    return pl.pallas_call(
        _stage2_kernel,
        grid=grid,
        in_specs=[
            pl.BlockSpec((tile, 384), row),
            pl.BlockSpec((tile, 384), row),
            pl.BlockSpec((1, h), fixed),
            pl.BlockSpec((h, h), fixed),
            pl.BlockSpec((1, h), fixed),
            pl.BlockSpec((h, h), fixed),
        ],
        out_specs=[
            pl.BlockSpec((tile, h), row),
            pl.BlockSpec((tile, h), row),
        ],
        out_shape=[
            jax.ShapeDtypeStruct((e, h), jnp.float32),
            jax.ShapeDtypeStruct((e, h), jnp.float32),
        ],
    )(gs, gd, b2_on[None, :], w3_on, b3_on[None, :], w3_off)


def _stage3_kernel(b3_ref, z3s_ref, z3d_ref, ws_ref, wd_ref, part_ref):
    i = pl.program_id(0)
    b3 = b3_ref[...]
    ws = ws_ref[...]
    wd = wd_ref[...]
    v0 = (b3 + z3s_ref[...]) * ws
    v1 = (b3 + z3d_ref[...]) * wd
    g3w = (jax.nn.relu(v0) + jax.nn.relu(v1)) * ((ws + wd) * 0.5)
    psum = jnp.sum(g3w.reshape(-1, 8, 128), axis=0)

    @pl.when(i == 0)
    def _():
        part_ref[...] = jnp.zeros_like(part_ref)
    part_ref[...] += psum


def _stage3(b3, z3s, z3d, gs, gd, tile=2560):
    e, h = b3.shape
    grid = (e // tile,)
    row = lambda i: (i, 0)
    wcol = lambda i: (i, 2)
    return pl.pallas_call(
        _stage3_kernel,
        grid=grid,
        in_specs=[
            pl.BlockSpec((tile, h), row),
            pl.BlockSpec((tile, h), row),
            pl.BlockSpec((tile, h), row),
            pl.BlockSpec((tile, h), wcol),
            pl.BlockSpec((tile, h), wcol),
        ],
        out_specs=pl.BlockSpec((8, h), lambda i: (0, 0)),
        out_shape=jax.ShapeDtypeStruct((8, h), jnp.float32),
    )(b3, z3s, z3d, gs, gd)


def kernel(x, edge_index, batch, w1_on, b1_on, w1_off, b1_off, w2_on, b2_on,
           w2_off, b2_off, w3_on, b3_on, w3_off, b3_off, wl1, bl1, wl2, bl2):
    n = x.shape[0]

    ei_flat = edge_index.reshape(-1)
    degp = _make_deg_kernel()(ei_flat)
    deg = degp[:N] + degp[NACC:NACC + N]
    invdeg = 1.0 / jnp.maximum(deg, 1.0)

    U, V = _dual_matmul(x, w1_on, b1_on, w1_off, b1_off)
    Q1h, R1 = _p1q1r1(U, V, invdeg, w2_on, w2_off)

    accp = _make_nbr_kernel()(ei_flat, R1)
    acc = accp[:N] + accp[NACC:NACC + N]
    z2 = deg * b2_off + (deg * R1 + acc) * 0.5

    packed = jnp.concatenate([Q1h, z2, invdeg], axis=1)
    Gs, Gd = _make_gathp_kernel()(ei_flat, packed)
    A3, B3 = _stage2(Gs, Gd, b2_on, w3_on, b3_on, w3_off)

    z3p = _make_a3scat_kernel()(ei_flat, A3)
    z3 = z3p[:N] + z3p[NACC:NACC + N] + deg * b3_off
    z3s, z3d = _make_gath2_kernel()(ei_flat, z3)

    parts = _stage3(B3, z3s, z3d, Gs, Gd)
    total = jnp.sum(parts, axis=0)
    g = (total / n)[None, :]

    h1 = jax.nn.relu(g @ wl1 + bl1)
    logits = h1 @ wl2 + bl2
    return jax.nn.log_softmax(logits, axis=-1)
